# baseline (speedup 1.0000x reference)
import jax
import jax.numpy as jnp
from jax.experimental import pallas as pl

N = 10000
NF = 128
HC = 64
H = 8
NC = 64


def _bias_kernel(x_ref, b_ref, o_ref):
    o_ref[...] = x_ref[...] + b_ref[...]


def _bias_add(x, b):
    return pl.pallas_call(
        _bias_kernel,
        out_shape=jax.ShapeDtypeStruct(x.shape, x.dtype),
    )(x, b[None, :])


def _gat(x, src, dst, W, att_src, att_dst, bias, heads, out_ch, concat, n):
    h = (x @ W).reshape(n, heads, out_ch)
    alpha_src = jnp.sum(h * att_src[None, :, :], axis=-1)
    alpha_dst = jnp.sum(h * att_dst[None, :, :], axis=-1)
    alpha = alpha_src[src] + alpha_dst[dst]
    alpha = jax.nn.leaky_relu(alpha, 0.2)
    amax = jax.ops.segment_max(alpha, dst, num_segments=n)
    alpha = jnp.exp(alpha - amax[dst])
    denom = jax.ops.segment_sum(alpha, dst, num_segments=n)
    alpha = alpha / (denom[dst] + 1e-16)
    msg = h[src] * alpha[:, :, None]
    out = jax.ops.segment_sum(msg, dst, num_segments=n)
    if concat:
        out = out.reshape(n, heads * out_ch)
    else:
        out = out.mean(axis=1)
    return _bias_add(out, bias)


def kernel(x, edge_index, W1, att_src1, att_dst1, b1, W2, att_src2, att_dst2, b2):
    n = x.shape[0]
    src = edge_index[0]
    dst = edge_index[1]
    loop = jnp.arange(n, dtype=src.dtype)
    src = jnp.concatenate([src, loop])
    dst = jnp.concatenate([dst, loop])
    emb = _gat(x, src, dst, W1, att_src1, att_dst1, b1, H, HC, True, n)
    h2 = jax.nn.elu(emb)
    logits = _gat(h2, src, dst, W2, att_src2, att_dst2, b2, 1, NC, False, n)
    return (logits, emb)


# speedup vs baseline: 19.1887x; 19.1887x over previous
import dataclasses
import functools

import jax
import jax.numpy as jnp
from jax import lax
from jax.experimental import pallas as pl
from jax.experimental.pallas import tpu as pltpu
from jax.experimental.pallas import tpu_sc as plsc

N = 10000
NF = 128
HC = 64
H = 8
NC = 64
N1 = 10240
E0 = 320000
ET = 331776
RPW = N1 // 16
C1 = 128
C2 = 128

F32 = jnp.float32
I32 = jnp.int32

_HIGH = jax.lax.Precision.HIGHEST


def _sc_compiler_params():
    cp = pltpu.CompilerParams()
    fields = pltpu.CompilerParams.__dataclass_fields__
    if "needs_layout_passes" in fields:
        cp = dataclasses.replace(cp, needs_layout_passes=False)
    if "use_tc_tiling_on_sc" in fields:
        cp = dataclasses.replace(cp, use_tc_tiling_on_sc=False)
    return cp


def _tc_prep_body(x_ref, w1_ref, as_ref, ad_ref, tab_ref, an_ref, shift_ref, mx_ref):
    i = pl.program_id(0)

    @pl.when(i == 0)
    def _():
        mx_ref[0] = -1e30
        mx_ref[1] = -1e30

    h = lax.dot_general(x_ref[...], w1_ref[...], (((1,), (0,)), ((), ())),
                        precision=_HIGH, preferred_element_type=F32)
    hr = h.reshape(h.shape[0], H, HC)
    a_s = jnp.sum(hr * as_ref[...][None], axis=-1)
    a_d = jnp.sum(hr * ad_ref[...][None], axis=-1)
    for p in range(4):
        tab_ref[p] = h[:, p * 128:(p + 1) * 128]
    an_ref[...] = jnp.concatenate([a_s, a_d], axis=1)
    mx_ref[0] = jnp.maximum(mx_ref[0], jnp.max(a_s))
    mx_ref[1] = jnp.maximum(mx_ref[1], jnp.max(a_d))
    shift_ref[...] = jnp.full((8, 16), jnp.maximum(mx_ref[0] + mx_ref[1], 0.0), F32)


def _tc_prep(x_p, W1, att_src1, att_dst1):
    BN = 128
    grid = (N1 // BN,)
    return pl.pallas_call(
        _tc_prep_body,
        grid=grid,
        in_specs=[
            pl.BlockSpec((BN, NF), lambda i: (i, 0)),
            pl.BlockSpec((NF, H * HC), lambda i: (0, 0)),
            pl.BlockSpec((H, HC), lambda i: (0, 0)),
            pl.BlockSpec((H, HC), lambda i: (0, 0)),
        ],
        out_specs=[
            pl.BlockSpec((4, BN, 128), lambda i: (0, i, 0)),
            pl.BlockSpec((BN, 16), lambda i: (i, 0)),
            pl.BlockSpec((8, 16), lambda i: (0, 0)),
        ],
        out_shape=[
            jax.ShapeDtypeStruct((4, N1, 128), F32),
            jax.ShapeDtypeStruct((N1, 16), F32),
            jax.ShapeDtypeStruct((8, 16), F32),
        ],
        scratch_shapes=[pltpu.SMEM((2,), F32)],
    )(x_p, W1, att_src1, att_dst1)


def _sc1_body(tab_hbm, an_hbm, shift_hbm, src_hbm, dst_hbm,
              msg_out, den_out,
              shift_v, src_v, dst_v, idx_v, asg_v, adg_v,
              w16_v, msg_v, zbuf, zbuf16,
              acc_msg, acc_den):
    c = lax.axis_index("c")
    s = lax.axis_index("s")
    epw = ET // 16
    nchunks = epw // C1

    pltpu.sync_copy(shift_hbm, shift_v)
    shift = shift_v[...]

    @pl.loop(0, 32)
    def _(i):
        for k in range(8):
            zbuf[i, pl.ds(k * 16, 16)] = jnp.zeros((16,), F32)
        zbuf16[i, pl.ds(0, 16)] = jnp.zeros((16,), F32)

    r0 = s * RPW
    for p in range(2):
        P = 2 * c + p
        h0 = 4 * c + 2 * p

        for k in range(RPW // 32):
            pltpu.sync_copy(zbuf, acc_msg.at[pl.ds(r0 + k * 32, 32)])
            if p == 0:
                pltpu.sync_copy(zbuf16, acc_den.at[pl.ds(r0 + k * 32, 32)])

        @pl.loop(0, C1)
        def _(i):
            w16_v[i, pl.ds(0, 16)] = jnp.zeros((16,), F32)

        plsc.subcore_barrier()

        c0 = 2 * p

        @pl.loop(0, nchunks)
        def _(t):
            base = s * epw + t * C1
            pltpu.sync_copy(src_hbm.at[pl.ds(base, C1)], src_v)
            pltpu.sync_copy(dst_hbm.at[pl.ds(base, C1)], dst_v)
            pltpu.sync_copy(an_hbm.at[src_v], asg_v)
            pltpu.sync_copy(an_hbm.at[dst_v], adg_v)

            @pl.loop(0, C1, step=16)
            def _(g):
                rows = lax.iota(I32, 16) + g
                cs = jnp.full((16,), h0, I32)
                cd = jnp.full((16,), 8 + h0, I32)
                a0 = (plsc.load_gather(asg_v, [rows, cs])
                      + plsc.load_gather(adg_v, [rows, cd]))
                a0 = jnp.maximum(a0, 0.0) + 0.2 * jnp.minimum(a0, 0.0)
                w0 = jnp.exp(a0 - shift)
                a1 = (plsc.load_gather(asg_v, [rows, cs + 1])
                      + plsc.load_gather(adg_v, [rows, cd + 1]))
                a1 = jnp.maximum(a1, 0.0) + 0.2 * jnp.minimum(a1, 0.0)
                w1 = jnp.exp(a1 - shift)
                plsc.store_scatter(w16_v, [rows, jnp.full((16,), c0, I32)], w0)
                plsc.store_scatter(w16_v, [rows, jnp.full((16,), c0 + 1, I32)], w1)
                idx_v[pl.ds(g, 16)] = src_v[pl.ds(g, 16)] + P * N1

            pltpu.sync_copy(tab_hbm.at[idx_v], msg_v)

            @pl.loop(0, C1)
            def _(i):
                wrow = w16_v[i, pl.ds(0, 16)]
                v0 = jnp.full((16,), wrow[c0], F32)
                v1 = jnp.full((16,), wrow[c0 + 1], F32)
                for k in range(4):
                    msg_v[i, pl.ds(k * 16, 16)] = msg_v[i, pl.ds(k * 16, 16)] * v0
                for k in range(4, 8):
                    msg_v[i, pl.ds(k * 16, 16)] = msg_v[i, pl.ds(k * 16, 16)] * v1

            pltpu.sync_copy(msg_v, acc_msg.at[dst_v], add=True)
            pltpu.sync_copy(w16_v, acc_den.at[dst_v], add=True)

        plsc.subcore_barrier()
        pltpu.sync_copy(acc_msg.at[pl.ds(r0, RPW)],
                        msg_out.at[pl.ds(P * N1 + r0, RPW)])
        if p == 1:
            pltpu.sync_copy(acc_den.at[pl.ds(r0, RPW)],
                            den_out.at[pl.ds(c * N1 + r0, RPW)])
        plsc.subcore_barrier()


def _sc_layer1(tab1f, a_nodes, shift_rep, srcE, dstE):
    mesh = plsc.VectorSubcoreMesh(core_axis_name="c", subcore_axis_name="s")
    fn = functools.partial(
        pl.kernel,
        out_type=[
            jax.ShapeDtypeStruct((4 * N1, 128), F32),
            jax.ShapeDtypeStruct((2 * N1, 16), F32),
        ],
        mesh=mesh,
        scratch_types=[
            pltpu.VMEM((16,), F32),
            pltpu.VMEM((C1,), I32),
            pltpu.VMEM((C1,), I32),
            pltpu.VMEM((C1,), I32),
            pltpu.VMEM((C1, 16), F32),
            pltpu.VMEM((C1, 16), F32),
            pltpu.VMEM((C1, 16), F32),
            pltpu.VMEM((C1, 128), F32),
            pltpu.VMEM((32, 128), F32),
            pltpu.VMEM((32, 16), F32),
            pltpu.VMEM_SHARED((N1, 128), F32),
            pltpu.VMEM_SHARED((N1, 16), F32),
        ],
        compiler_params=_sc_compiler_params(),
    )(_sc1_body)
    return fn(tab1f, a_nodes, shift_rep, srcE, dstE)


def _tc_mid_body(msg_ref, den_ref, b1_ref, w2_ref, as2_ref, ad2_ref,
                 emb_ref, tab2_ref, a2n_ref, shift2_ref, mx_ref):
    i = pl.program_id(0)

    @pl.when(i == 0)
    def _():
        mx_ref[0] = -1e30
        mx_ref[1] = -1e30

    segs = []
    for h in range(H):
        seg = msg_ref[h // 2, :, (h % 2) * 64:(h % 2 + 1) * 64]
        dcol = den_ref[h // 4, :, (h % 4):(h % 4) + 1]
        segs.append(seg / (dcol + 1e-16))
    emb = jnp.concatenate(segs, axis=1) + b1_ref[...]
    emb_ref[...] = emb
    h2 = jnp.where(emb > 0, emb, jnp.exp(emb) - 1.0)
    h2m = lax.dot_general(h2, w2_ref[...], (((1,), (0,)), ((), ())),
                          precision=_HIGH, preferred_element_type=F32)
    tab2_ref[...] = h2m
    a_s = jnp.sum(h2m * as2_ref[...], axis=-1, keepdims=True)
    a_d = jnp.sum(h2m * ad2_ref[...], axis=-1, keepdims=True)
    col = lax.broadcasted_iota(I32, (a_s.shape[0], 16), 1)
    a2n_ref[...] = jnp.where(col == 0, a_s, jnp.where(col == 1, a_d, 0.0))
    mx_ref[0] = jnp.maximum(mx_ref[0], jnp.max(a_s))
    mx_ref[1] = jnp.maximum(mx_ref[1], jnp.max(a_d))
    shift2_ref[...] = jnp.full((8, 16), jnp.maximum(mx_ref[0] + mx_ref[1], 0.0), F32)


def _tc_mid(msg1, den1, b1_2d, W2, att_src2, att_dst2):
    BN = 256
    grid = (N1 // BN,)
    return pl.pallas_call(
        _tc_mid_body,
        grid=grid,
        in_specs=[
            pl.BlockSpec((4, BN, 128), lambda i: (0, i, 0)),
            pl.BlockSpec((2, BN, 16), lambda i: (0, i, 0)),
            pl.BlockSpec((1, H * HC), lambda i: (0, 0)),
            pl.BlockSpec((H * HC, NC), lambda i: (0, 0)),
            pl.BlockSpec((1, NC), lambda i: (0, 0)),
            pl.BlockSpec((1, NC), lambda i: (0, 0)),
        ],
        out_specs=[
            pl.BlockSpec((BN, H * HC), lambda i: (i, 0)),
            pl.BlockSpec((BN, NC), lambda i: (i, 0)),
            pl.BlockSpec((BN, 16), lambda i: (i, 0)),
            pl.BlockSpec((8, 16), lambda i: (0, 0)),
        ],
        out_shape=[
            jax.ShapeDtypeStruct((N1, H * HC), F32),
            jax.ShapeDtypeStruct((N1, NC), F32),
            jax.ShapeDtypeStruct((N1, 16), F32),
            jax.ShapeDtypeStruct((8, 16), F32),
        ],
        scratch_shapes=[pltpu.SMEM((2,), F32)],
    )(msg1, den1, b1_2d, W2, att_src2, att_dst2)


def _sc2_body(tab_hbm, an_hbm, shift_hbm, src_hbm, dst_hbm,
              msg_out, den_out,
              shift_v, src_v, dst_v, asg_v, adg_v, w16_v, msg_v, zbuf, zbuf16,
              acc_msg, acc_den):
    c = lax.axis_index("c")
    s = lax.axis_index("s")
    epw = ET // 32
    nchunks = epw // C2

    pltpu.sync_copy(shift_hbm, shift_v)
    shift = shift_v[...]

    @pl.loop(0, 32)
    def _(i):
        for k in range(4):
            zbuf[i, pl.ds(k * 16, 16)] = jnp.zeros((16,), F32)
        zbuf16[i, pl.ds(0, 16)] = jnp.zeros((16,), F32)

    @pl.loop(0, C2)
    def _(i):
        w16_v[i, pl.ds(0, 16)] = jnp.zeros((16,), F32)

    r0 = s * RPW
    for k in range(RPW // 32):
        pltpu.sync_copy(zbuf, acc_msg.at[pl.ds(r0 + k * 32, 32)])
        pltpu.sync_copy(zbuf16, acc_den.at[pl.ds(r0 + k * 32, 32)])

    plsc.subcore_barrier()

    @pl.loop(0, nchunks)
    def _(t):
        base = c * (ET // 2) + s * epw + t * C2
        pltpu.sync_copy(src_hbm.at[pl.ds(base, C2)], src_v)
        pltpu.sync_copy(dst_hbm.at[pl.ds(base, C2)], dst_v)
        pltpu.sync_copy(an_hbm.at[src_v], asg_v)
        pltpu.sync_copy(an_hbm.at[dst_v], adg_v)

        @pl.loop(0, C2, step=16)
        def _(g):
            rows = lax.iota(I32, 16) + g
            a0 = (plsc.load_gather(asg_v, [rows, jnp.full((16,), 0, I32)])
                  + plsc.load_gather(adg_v, [rows, jnp.full((16,), 1, I32)]))
            a0 = jnp.maximum(a0, 0.0) + 0.2 * jnp.minimum(a0, 0.0)
            w0 = jnp.exp(a0 - shift)
            plsc.store_scatter(w16_v, [rows, jnp.full((16,), 0, I32)], w0)

        pltpu.sync_copy(tab_hbm.at[src_v], msg_v)

        @pl.loop(0, C2)
        def _(i):
            wrow = w16_v[i, pl.ds(0, 16)]
            v0 = jnp.full((16,), wrow[0], F32)
            for k in range(4):
                msg_v[i, pl.ds(k * 16, 16)] = msg_v[i, pl.ds(k * 16, 16)] * v0

        pltpu.sync_copy(msg_v, acc_msg.at[dst_v], add=True)
        pltpu.sync_copy(w16_v, acc_den.at[dst_v], add=True)

    plsc.subcore_barrier()
    pltpu.sync_copy(acc_msg.at[pl.ds(r0, RPW)],
                    msg_out.at[pl.ds(c * N1 + r0, RPW)])
    pltpu.sync_copy(acc_den.at[pl.ds(r0, RPW)],
                    den_out.at[pl.ds(c * N1 + r0, RPW)])


def _sc_layer2(tab2, a2_nodes, shift2_rep, srcE, dstE):
    mesh = plsc.VectorSubcoreMesh(core_axis_name="c", subcore_axis_name="s")
    fn = functools.partial(
        pl.kernel,
        out_type=[
            jax.ShapeDtypeStruct((2 * N1, 64), F32),
            jax.ShapeDtypeStruct((2 * N1, 16), F32),
        ],
        mesh=mesh,
        scratch_types=[
            pltpu.VMEM((16,), F32),
            pltpu.VMEM((C2,), I32),
            pltpu.VMEM((C2,), I32),
            pltpu.VMEM((C2, 16), F32),
            pltpu.VMEM((C2, 16), F32),
            pltpu.VMEM((C2, 16), F32),
            pltpu.VMEM((C2, 64), F32),
            pltpu.VMEM((32, 64), F32),
            pltpu.VMEM((32, 16), F32),
            pltpu.VMEM_SHARED((N1, 64), F32),
            pltpu.VMEM_SHARED((N1, 16), F32),
        ],
        compiler_params=_sc_compiler_params(),
    )(_sc2_body)
    return fn(tab2, a2_nodes, shift2_rep, srcE, dstE)


def _tc_final_body(msg_ref, den_ref, b2_ref, out_ref):
    d = den_ref[0, :, 0:1] + den_ref[1, :, 0:1] + 1e-16
    out_ref[...] = (msg_ref[0] + msg_ref[1]) / d + b2_ref[...]


def _tc_final(msg2, den2, b2_2d):
    BN = 512
    grid = (N1 // BN,)
    return pl.pallas_call(
        _tc_final_body,
        grid=grid,
        in_specs=[
            pl.BlockSpec((2, BN, NC), lambda i: (0, i, 0)),
            pl.BlockSpec((2, BN, 16), lambda i: (0, i, 0)),
            pl.BlockSpec((1, NC), lambda i: (0, 0)),
        ],
        out_specs=pl.BlockSpec((BN, NC), lambda i: (i, 0)),
        out_shape=jax.ShapeDtypeStruct((N1, NC), F32),
    )(msg2, den2, b2_2d)


def kernel(x, edge_index, W1, att_src1, att_dst1, b1, W2, att_src2, att_dst2, b2):
    src = edge_index[0]
    dst = edge_index[1]
    loop = jnp.arange(N, dtype=src.dtype)
    fill = jnp.full((ET - E0 - N,), N, dtype=src.dtype)
    srcE = jnp.concatenate([src, loop, fill])
    dstE = jnp.concatenate([dst, loop, fill])

    x_p = jnp.pad(x, ((0, N1 - N), (0, 0)))

    tab1, a_nodes, shift = _tc_prep(x_p, W1, att_src1, att_dst1)
    tab1f = tab1.reshape(4 * N1, 128)
    shift_rep = shift[0]

    msg1, den1 = _sc_layer1(tab1f, a_nodes, shift_rep, srcE, dstE)
    msg1 = msg1.reshape(4, N1, 128)
    den1 = den1.reshape(2, N1, 16)

    emb_p, tab2, a2_nodes, shift2 = _tc_mid(msg1, den1, b1.reshape(1, H * HC),
                                            W2, att_src2, att_dst2)
    msg2, den2 = _sc_layer2(tab2, a2_nodes, shift2[0], srcE, dstE)
    logits_p = _tc_final(msg2.reshape(2, N1, 64), den2.reshape(2, N1, 16),
                         b2.reshape(1, NC))
    return (logits_p[:N], emb_p[:N])


# speedup vs baseline: 26.6973x; 1.3913x over previous
import dataclasses
import functools

import jax
import jax.numpy as jnp
from jax import lax
from jax.experimental import pallas as pl
from jax.experimental.pallas import tpu as pltpu
from jax.experimental.pallas import tpu_sc as plsc

N = 10000
NF = 128
HC = 64
H = 8
NC = 64
N1 = 10240
E0 = 320000
ET = 331776
RPW = N1 // 16
C1 = 128
C2 = 128

F32 = jnp.float32
I32 = jnp.int32

_HIGH = jax.lax.Precision.HIGHEST


def _sc_compiler_params():
    cp = pltpu.CompilerParams()
    fields = pltpu.CompilerParams.__dataclass_fields__
    if "needs_layout_passes" in fields:
        cp = dataclasses.replace(cp, needs_layout_passes=False)
    if "use_tc_tiling_on_sc" in fields:
        cp = dataclasses.replace(cp, use_tc_tiling_on_sc=False)
    return cp


def _tc_prep_body(x_ref, w1_ref, as_ref, ad_ref, tab_ref, an_ref, shift_ref, mx_ref):
    i = pl.program_id(0)

    @pl.when(i == 0)
    def _():
        mx_ref[0] = -1e30
        mx_ref[1] = -1e30

    h = lax.dot_general(x_ref[...], w1_ref[...], (((1,), (0,)), ((), ())),
                        precision=_HIGH, preferred_element_type=F32)
    hr = h.reshape(h.shape[0], H, HC)
    a_s = jnp.sum(hr * as_ref[...][None], axis=-1)
    a_d = jnp.sum(hr * ad_ref[...][None], axis=-1)
    for p in range(4):
        tab_ref[p] = h[:, p * 128:(p + 1) * 128]
    an_ref[...] = jnp.concatenate([a_s, a_d], axis=1)
    mx_ref[0] = jnp.maximum(mx_ref[0], jnp.max(a_s))
    mx_ref[1] = jnp.maximum(mx_ref[1], jnp.max(a_d))
    shift_ref[...] = jnp.full((8, 16), jnp.maximum(mx_ref[0] + mx_ref[1], 0.0), F32)


def _tc_prep(x_p, W1, att_src1, att_dst1):
    BN = 128
    grid = (N1 // BN,)
    return pl.pallas_call(
        _tc_prep_body,
        grid=grid,
        in_specs=[
            pl.BlockSpec((BN, NF), lambda i: (i, 0)),
            pl.BlockSpec((NF, H * HC), lambda i: (0, 0)),
            pl.BlockSpec((H, HC), lambda i: (0, 0)),
            pl.BlockSpec((H, HC), lambda i: (0, 0)),
        ],
        out_specs=[
            pl.BlockSpec((4, BN, 128), lambda i: (0, i, 0)),
            pl.BlockSpec((BN, 16), lambda i: (i, 0)),
            pl.BlockSpec((8, 16), lambda i: (0, 0)),
        ],
        out_shape=[
            jax.ShapeDtypeStruct((4, N1, 128), F32),
            jax.ShapeDtypeStruct((N1, 16), F32),
            jax.ShapeDtypeStruct((8, 16), F32),
        ],
        scratch_shapes=[pltpu.SMEM((2,), F32)],
    )(x_p, W1, att_src1, att_dst1)


def _sc1_body(tab_hbm, an_hbm, shift_hbm, src_hbm, dst_hbm,
              msg_out, den_out,
              shift_v, src_v, dst_v, idx_v, asg_v, adg_v,
              w16_v, msg_v, zbuf, zbuf16,
              acc_msg, acc_den, sem):
    c = lax.axis_index("c")
    s = lax.axis_index("s")
    epw = ET // 16
    nchunks = epw // C1

    pltpu.sync_copy(shift_hbm, shift_v)
    shift = shift_v[...]

    @pl.loop(0, 32)
    def _(i):
        for k in range(8):
            zbuf[i, pl.ds(k * 16, 16)] = jnp.zeros((16,), F32)
        zbuf16[i, pl.ds(0, 16)] = jnp.zeros((16,), F32)

    r0 = s * RPW
    for p in range(2):
        P = 2 * c + p
        h0 = 4 * c + 2 * p

        for k in range(RPW // 32):
            pltpu.sync_copy(zbuf, acc_msg.at[pl.ds(r0 + k * 32, 32)])
            if p == 0:
                pltpu.sync_copy(zbuf16, acc_den.at[pl.ds(r0 + k * 32, 32)])

        @pl.loop(0, C1)
        def _(i):
            w16_v[i, pl.ds(0, 16)] = jnp.zeros((16,), F32)

        plsc.subcore_barrier()

        c0 = 2 * p

        @pl.loop(0, nchunks)
        def _(t):
            base = s * epw + t * C1
            pltpu.sync_copy(src_hbm.at[pl.ds(base, C1)], src_v)
            pltpu.sync_copy(dst_hbm.at[pl.ds(base, C1)], dst_v)

            @plsc.parallel_loop(0, C1, step=16, unroll=2)
            def _(g):
                idx_v[pl.ds(g, 16)] = src_v[pl.ds(g, 16)] + P * N1

            cp = pltpu.async_copy(tab_hbm.at[idx_v], msg_v, sem)
            pltpu.sync_copy(an_hbm.at[src_v], asg_v)
            pltpu.sync_copy(an_hbm.at[dst_v], adg_v)

            @plsc.parallel_loop(0, C1, step=16, unroll=2)
            def _(g):
                rows = lax.iota(I32, 16) + g
                cs = jnp.full((16,), h0, I32)
                cd = jnp.full((16,), 8 + h0, I32)
                a0 = (plsc.load_gather(asg_v, [rows, cs])
                      + plsc.load_gather(adg_v, [rows, cd]))
                a0 = jnp.maximum(a0, 0.0) + 0.2 * jnp.minimum(a0, 0.0)
                w0 = jnp.exp(a0 - shift)
                a1 = (plsc.load_gather(asg_v, [rows, cs + 1])
                      + plsc.load_gather(adg_v, [rows, cd + 1]))
                a1 = jnp.maximum(a1, 0.0) + 0.2 * jnp.minimum(a1, 0.0)
                w1 = jnp.exp(a1 - shift)
                plsc.store_scatter(w16_v, [rows, jnp.full((16,), c0, I32)], w0)
                plsc.store_scatter(w16_v, [rows, jnp.full((16,), c0 + 1, I32)], w1)

            cp.wait()

            @plsc.parallel_loop(0, C1, unroll=4)
            def _(i):
                wrow = w16_v[i, pl.ds(0, 16)]
                v0 = jnp.full((16,), wrow[c0], F32)
                v1 = jnp.full((16,), wrow[c0 + 1], F32)
                for k in range(4):
                    msg_v[i, pl.ds(k * 16, 16)] = msg_v[i, pl.ds(k * 16, 16)] * v0
                for k in range(4, 8):
                    msg_v[i, pl.ds(k * 16, 16)] = msg_v[i, pl.ds(k * 16, 16)] * v1

            pltpu.sync_copy(msg_v, acc_msg.at[dst_v], add=True)
            pltpu.sync_copy(w16_v, acc_den.at[dst_v], add=True)

        plsc.subcore_barrier()
        pltpu.sync_copy(acc_msg.at[pl.ds(r0, RPW)],
                        msg_out.at[pl.ds(P * N1 + r0, RPW)])
        if p == 1:
            pltpu.sync_copy(acc_den.at[pl.ds(r0, RPW)],
                            den_out.at[pl.ds(c * N1 + r0, RPW)])
        plsc.subcore_barrier()


def _sc_layer1(tab1f, a_nodes, shift_rep, srcE, dstE):
    mesh = plsc.VectorSubcoreMesh(core_axis_name="c", subcore_axis_name="s")
    fn = functools.partial(
        pl.kernel,
        out_type=[
            jax.ShapeDtypeStruct((4 * N1, 128), F32),
            jax.ShapeDtypeStruct((2 * N1, 16), F32),
        ],
        mesh=mesh,
        scratch_types=[
            pltpu.VMEM((16,), F32),
            pltpu.VMEM((C1,), I32),
            pltpu.VMEM((C1,), I32),
            pltpu.VMEM((C1,), I32),
            pltpu.VMEM((C1, 16), F32),
            pltpu.VMEM((C1, 16), F32),
            pltpu.VMEM((C1, 16), F32),
            pltpu.VMEM((C1, 128), F32),
            pltpu.VMEM((32, 128), F32),
            pltpu.VMEM((32, 16), F32),
            pltpu.VMEM_SHARED((N1, 128), F32),
            pltpu.VMEM_SHARED((N1, 16), F32),
            pltpu.SemaphoreType.DMA,
        ],
        compiler_params=_sc_compiler_params(),
    )(_sc1_body)
    return fn(tab1f, a_nodes, shift_rep, srcE, dstE)


def _tc_mid_body(msg_ref, den_ref, b1_ref, w2_ref, as2_ref, ad2_ref,
                 emb_ref, tab2_ref, a2n_ref, shift2_ref, mx_ref):
    i = pl.program_id(0)

    @pl.when(i == 0)
    def _():
        mx_ref[0] = -1e30
        mx_ref[1] = -1e30

    segs = []
    for h in range(H):
        seg = msg_ref[h // 2, :, (h % 2) * 64:(h % 2 + 1) * 64]
        dcol = den_ref[h // 4, :, (h % 4):(h % 4) + 1]
        segs.append(seg / (dcol + 1e-16))
    emb = jnp.concatenate(segs, axis=1) + b1_ref[...]
    emb_ref[...] = emb
    h2 = jnp.where(emb > 0, emb, jnp.exp(emb) - 1.0)
    h2m = lax.dot_general(h2, w2_ref[...], (((1,), (0,)), ((), ())),
                          precision=_HIGH, preferred_element_type=F32)
    tab2_ref[...] = h2m
    a_s = jnp.sum(h2m * as2_ref[...], axis=-1, keepdims=True)
    a_d = jnp.sum(h2m * ad2_ref[...], axis=-1, keepdims=True)
    col = lax.broadcasted_iota(I32, (a_s.shape[0], 16), 1)
    a2n_ref[...] = jnp.where(col == 0, a_s, jnp.where(col == 1, a_d, 0.0))
    mx_ref[0] = jnp.maximum(mx_ref[0], jnp.max(a_s))
    mx_ref[1] = jnp.maximum(mx_ref[1], jnp.max(a_d))
    shift2_ref[...] = jnp.full((8, 16), jnp.maximum(mx_ref[0] + mx_ref[1], 0.0), F32)


def _tc_mid(msg1, den1, b1_2d, W2, att_src2, att_dst2):
    BN = 256
    grid = (N1 // BN,)
    return pl.pallas_call(
        _tc_mid_body,
        grid=grid,
        in_specs=[
            pl.BlockSpec((4, BN, 128), lambda i: (0, i, 0)),
            pl.BlockSpec((2, BN, 16), lambda i: (0, i, 0)),
            pl.BlockSpec((1, H * HC), lambda i: (0, 0)),
            pl.BlockSpec((H * HC, NC), lambda i: (0, 0)),
            pl.BlockSpec((1, NC), lambda i: (0, 0)),
            pl.BlockSpec((1, NC), lambda i: (0, 0)),
        ],
        out_specs=[
            pl.BlockSpec((BN, H * HC), lambda i: (i, 0)),
            pl.BlockSpec((BN, NC), lambda i: (i, 0)),
            pl.BlockSpec((BN, 16), lambda i: (i, 0)),
            pl.BlockSpec((8, 16), lambda i: (0, 0)),
        ],
        out_shape=[
            jax.ShapeDtypeStruct((N1, H * HC), F32),
            jax.ShapeDtypeStruct((N1, NC), F32),
            jax.ShapeDtypeStruct((N1, 16), F32),
            jax.ShapeDtypeStruct((8, 16), F32),
        ],
        scratch_shapes=[pltpu.SMEM((2,), F32)],
    )(msg1, den1, b1_2d, W2, att_src2, att_dst2)


def _sc2_body(tab_hbm, an_hbm, shift_hbm, src_hbm, dst_hbm,
              msg_out, den_out,
              shift_v, src_v, dst_v, asg_v, adg_v, w16_v, msg_v, zbuf, zbuf16,
              acc_msg, acc_den, sem):
    c = lax.axis_index("c")
    s = lax.axis_index("s")
    epw = ET // 32
    nchunks = epw // C2

    pltpu.sync_copy(shift_hbm, shift_v)
    shift = shift_v[...]

    @pl.loop(0, 32)
    def _(i):
        for k in range(4):
            zbuf[i, pl.ds(k * 16, 16)] = jnp.zeros((16,), F32)
        zbuf16[i, pl.ds(0, 16)] = jnp.zeros((16,), F32)

    @pl.loop(0, C2)
    def _(i):
        w16_v[i, pl.ds(0, 16)] = jnp.zeros((16,), F32)

    r0 = s * RPW
    for k in range(RPW // 32):
        pltpu.sync_copy(zbuf, acc_msg.at[pl.ds(r0 + k * 32, 32)])
        pltpu.sync_copy(zbuf16, acc_den.at[pl.ds(r0 + k * 32, 32)])

    plsc.subcore_barrier()

    @pl.loop(0, nchunks)
    def _(t):
        base = c * (ET // 2) + s * epw + t * C2
        pltpu.sync_copy(src_hbm.at[pl.ds(base, C2)], src_v)
        pltpu.sync_copy(dst_hbm.at[pl.ds(base, C2)], dst_v)
        cp = pltpu.async_copy(tab_hbm.at[src_v], msg_v, sem)
        pltpu.sync_copy(an_hbm.at[src_v], asg_v)
        pltpu.sync_copy(an_hbm.at[dst_v], adg_v)

        @plsc.parallel_loop(0, C2, step=16, unroll=2)
        def _(g):
            rows = lax.iota(I32, 16) + g
            a0 = (plsc.load_gather(asg_v, [rows, jnp.full((16,), 0, I32)])
                  + plsc.load_gather(adg_v, [rows, jnp.full((16,), 1, I32)]))
            a0 = jnp.maximum(a0, 0.0) + 0.2 * jnp.minimum(a0, 0.0)
            w0 = jnp.exp(a0 - shift)
            plsc.store_scatter(w16_v, [rows, jnp.full((16,), 0, I32)], w0)

        cp.wait()

        @plsc.parallel_loop(0, C2, unroll=4)
        def _(i):
            wrow = w16_v[i, pl.ds(0, 16)]
            v0 = jnp.full((16,), wrow[0], F32)
            for k in range(4):
                msg_v[i, pl.ds(k * 16, 16)] = msg_v[i, pl.ds(k * 16, 16)] * v0

        pltpu.sync_copy(msg_v, acc_msg.at[dst_v], add=True)
        pltpu.sync_copy(w16_v, acc_den.at[dst_v], add=True)

    plsc.subcore_barrier()
    pltpu.sync_copy(acc_msg.at[pl.ds(r0, RPW)],
                    msg_out.at[pl.ds(c * N1 + r0, RPW)])
    pltpu.sync_copy(acc_den.at[pl.ds(r0, RPW)],
                    den_out.at[pl.ds(c * N1 + r0, RPW)])


def _sc_layer2(tab2, a2_nodes, shift2_rep, srcE, dstE):
    mesh = plsc.VectorSubcoreMesh(core_axis_name="c", subcore_axis_name="s")
    fn = functools.partial(
        pl.kernel,
        out_type=[
            jax.ShapeDtypeStruct((2 * N1, 64), F32),
            jax.ShapeDtypeStruct((2 * N1, 16), F32),
        ],
        mesh=mesh,
        scratch_types=[
            pltpu.VMEM((16,), F32),
            pltpu.VMEM((C2,), I32),
            pltpu.VMEM((C2,), I32),
            pltpu.VMEM((C2, 16), F32),
            pltpu.VMEM((C2, 16), F32),
            pltpu.VMEM((C2, 16), F32),
            pltpu.VMEM((C2, 64), F32),
            pltpu.VMEM((32, 64), F32),
            pltpu.VMEM((32, 16), F32),
            pltpu.VMEM_SHARED((N1, 64), F32),
            pltpu.VMEM_SHARED((N1, 16), F32),
            pltpu.SemaphoreType.DMA,
        ],
        compiler_params=_sc_compiler_params(),
    )(_sc2_body)
    return fn(tab2, a2_nodes, shift2_rep, srcE, dstE)


def _tc_final_body(msg_ref, den_ref, b2_ref, out_ref):
    d = den_ref[0, :, 0:1] + den_ref[1, :, 0:1] + 1e-16
    out_ref[...] = (msg_ref[0] + msg_ref[1]) / d + b2_ref[...]


def _tc_final(msg2, den2, b2_2d):
    BN = 512
    grid = (N1 // BN,)
    return pl.pallas_call(
        _tc_final_body,
        grid=grid,
        in_specs=[
            pl.BlockSpec((2, BN, NC), lambda i: (0, i, 0)),
            pl.BlockSpec((2, BN, 16), lambda i: (0, i, 0)),
            pl.BlockSpec((1, NC), lambda i: (0, 0)),
        ],
        out_specs=pl.BlockSpec((BN, NC), lambda i: (i, 0)),
        out_shape=jax.ShapeDtypeStruct((N1, NC), F32),
    )(msg2, den2, b2_2d)


def kernel(x, edge_index, W1, att_src1, att_dst1, b1, W2, att_src2, att_dst2, b2):
    src = edge_index[0]
    dst = edge_index[1]
    loop = jnp.arange(N, dtype=src.dtype)
    fill = jnp.full((ET - E0 - N,), N, dtype=src.dtype)
    srcE = jnp.concatenate([src, loop, fill])
    dstE = jnp.concatenate([dst, loop, fill])

    x_p = jnp.pad(x, ((0, N1 - N), (0, 0)))

    tab1, a_nodes, shift = _tc_prep(x_p, W1, att_src1, att_dst1)
    tab1f = tab1.reshape(4 * N1, 128)
    shift_rep = shift[0]

    msg1, den1 = _sc_layer1(tab1f, a_nodes, shift_rep, srcE, dstE)
    msg1 = msg1.reshape(4, N1, 128)
    den1 = den1.reshape(2, N1, 16)

    emb_p, tab2, a2_nodes, shift2 = _tc_mid(msg1, den1, b1.reshape(1, H * HC),
                                            W2, att_src2, att_dst2)
    msg2, den2 = _sc_layer2(tab2, a2_nodes, shift2[0], srcE, dstE)
    logits_p = _tc_final(msg2.reshape(2, N1, 64), den2.reshape(2, N1, 16),
                         b2.reshape(1, NC))
    return (logits_p[:N], emb_p[:N])


# speedup vs baseline: 31.2714x; 1.1713x over previous
import dataclasses
import functools

import jax
import jax.numpy as jnp
from jax import lax
from jax.experimental import pallas as pl
from jax.experimental.pallas import tpu as pltpu
from jax.experimental.pallas import tpu_sc as plsc

N = 10000
NF = 128
HC = 64
H = 8
NC = 64
N1 = 10240
E0 = 320000
ET = 331776
RPW = N1 // 16
C1 = 128
C2 = 128

F32 = jnp.float32
I32 = jnp.int32

_HIGH = jax.lax.Precision.HIGHEST


def _sc_compiler_params():
    cp = pltpu.CompilerParams()
    fields = pltpu.CompilerParams.__dataclass_fields__
    if "needs_layout_passes" in fields:
        cp = dataclasses.replace(cp, needs_layout_passes=False)
    if "use_tc_tiling_on_sc" in fields:
        cp = dataclasses.replace(cp, use_tc_tiling_on_sc=False)
    return cp


def _tc_prep_body(x_ref, w1_ref, as_ref, ad_ref, tab_ref, an_ref, shift_ref, mx_ref):
    i = pl.program_id(0)

    @pl.when(i == 0)
    def _():
        mx_ref[0] = -1e30
        mx_ref[1] = -1e30

    h = lax.dot_general(x_ref[...], w1_ref[...], (((1,), (0,)), ((), ())),
                        precision=_HIGH, preferred_element_type=F32)
    hr = h.reshape(h.shape[0], H, HC)
    a_s = jnp.sum(hr * as_ref[...][None], axis=-1)
    a_d = jnp.sum(hr * ad_ref[...][None], axis=-1)
    for p in range(4):
        tab_ref[p] = h[:, p * 128:(p + 1) * 128]
    an_ref[...] = jnp.concatenate([a_s, a_d], axis=1)
    mx_ref[0] = jnp.maximum(mx_ref[0], jnp.max(a_s))
    mx_ref[1] = jnp.maximum(mx_ref[1], jnp.max(a_d))
    shift_ref[...] = jnp.full((8, 16), jnp.maximum(mx_ref[0] + mx_ref[1], 0.0), F32)


def _tc_prep(x_p, W1, att_src1, att_dst1):
    BN = 128
    grid = (N1 // BN,)
    return pl.pallas_call(
        _tc_prep_body,
        grid=grid,
        in_specs=[
            pl.BlockSpec((BN, NF), lambda i: (i, 0)),
            pl.BlockSpec((NF, H * HC), lambda i: (0, 0)),
            pl.BlockSpec((H, HC), lambda i: (0, 0)),
            pl.BlockSpec((H, HC), lambda i: (0, 0)),
        ],
        out_specs=[
            pl.BlockSpec((4, BN, 128), lambda i: (0, i, 0)),
            pl.BlockSpec((BN, 16), lambda i: (i, 0)),
            pl.BlockSpec((8, 16), lambda i: (0, 0)),
        ],
        out_shape=[
            jax.ShapeDtypeStruct((4, N1, 128), F32),
            jax.ShapeDtypeStruct((N1, 16), F32),
            jax.ShapeDtypeStruct((8, 16), F32),
        ],
        scratch_shapes=[pltpu.SMEM((2,), F32)],
    )(x_p, W1, att_src1, att_dst1)


def _sc1_body(tab_hbm, an_hbm, shift_hbm, src_hbm, dst_hbm,
              msg_out, den_out,
              shift_v, src_v, dst_v, idx_v, asg_v, adg_v,
              w16_v, msg_v, zbuf, zbuf16,
              acc_msg, acc_den, sem, sem2):
    c = lax.axis_index("c")
    s = lax.axis_index("s")
    epw = ET // 16
    nchunks = epw // C1

    pltpu.sync_copy(shift_hbm, shift_v)
    shift = shift_v[...]

    @pl.loop(0, 32)
    def _(i):
        for k in range(8):
            zbuf[i, pl.ds(k * 16, 16)] = jnp.zeros((16,), F32)
        zbuf16[i, pl.ds(0, 16)] = jnp.zeros((16,), F32)

    r0 = s * RPW
    for p in range(2):
        P = 2 * c + p
        h0 = 4 * c + 2 * p

        for k in range(RPW // 32):
            pltpu.sync_copy(zbuf, acc_msg.at[pl.ds(r0 + k * 32, 32)])
            if p == 0:
                pltpu.sync_copy(zbuf16, acc_den.at[pl.ds(r0 + k * 32, 32)])

        @pl.loop(0, C1)
        def _(i):
            w16_v[i, pl.ds(0, 16)] = jnp.zeros((16,), F32)

        plsc.subcore_barrier()

        c0 = 2 * p

        @pl.loop(0, nchunks)
        def _(t):
            base = s * epw + t * C1
            h1 = pltpu.async_copy(src_hbm.at[pl.ds(base, C1)], src_v, sem2)
            h2 = pltpu.async_copy(dst_hbm.at[pl.ds(base, C1)], dst_v, sem2)
            h1.wait()
            h2.wait()

            @plsc.parallel_loop(0, C1, step=16, unroll=2)
            def _(g):
                idx_v[pl.ds(g, 16)] = src_v[pl.ds(g, 16)] + P * N1

            cp = pltpu.async_copy(tab_hbm.at[idx_v], msg_v, sem)
            h3 = pltpu.async_copy(an_hbm.at[src_v], asg_v, sem2)
            h4 = pltpu.async_copy(an_hbm.at[dst_v], adg_v, sem2)
            h3.wait()
            h4.wait()

            @plsc.parallel_loop(0, C1, step=16, unroll=2)
            def _(g):
                rows = lax.iota(I32, 16) + g
                cs = jnp.full((16,), h0, I32)
                cd = jnp.full((16,), 8 + h0, I32)
                a0 = (plsc.load_gather(asg_v, [rows, cs])
                      + plsc.load_gather(adg_v, [rows, cd]))
                a0 = jnp.maximum(a0, 0.0) + 0.2 * jnp.minimum(a0, 0.0)
                w0 = jnp.exp(a0 - shift)
                a1 = (plsc.load_gather(asg_v, [rows, cs + 1])
                      + plsc.load_gather(adg_v, [rows, cd + 1]))
                a1 = jnp.maximum(a1, 0.0) + 0.2 * jnp.minimum(a1, 0.0)
                w1 = jnp.exp(a1 - shift)
                plsc.store_scatter(w16_v, [rows, jnp.full((16,), c0, I32)], w0)
                plsc.store_scatter(w16_v, [rows, jnp.full((16,), c0 + 1, I32)], w1)

            cp.wait()

            @plsc.parallel_loop(0, C1, unroll=4)
            def _(i):
                wrow = w16_v[i, pl.ds(0, 16)]
                v0 = jnp.full((16,), wrow[c0], F32)
                v1 = jnp.full((16,), wrow[c0 + 1], F32)
                for k in range(4):
                    msg_v[i, pl.ds(k * 16, 16)] = msg_v[i, pl.ds(k * 16, 16)] * v0
                for k in range(4, 8):
                    msg_v[i, pl.ds(k * 16, 16)] = msg_v[i, pl.ds(k * 16, 16)] * v1

            h5 = pltpu.async_copy(msg_v, acc_msg.at[dst_v], sem2, add=True)
            h6 = pltpu.async_copy(w16_v, acc_den.at[dst_v], sem2, add=True)
            h5.wait()
            h6.wait()

        plsc.subcore_barrier()
        pltpu.sync_copy(acc_msg.at[pl.ds(r0, RPW)],
                        msg_out.at[pl.ds(P * N1 + r0, RPW)])
        if p == 1:
            pltpu.sync_copy(acc_den.at[pl.ds(r0, RPW)],
                            den_out.at[pl.ds(c * N1 + r0, RPW)])
        plsc.subcore_barrier()


def _sc_layer1(tab1f, a_nodes, shift_rep, srcE, dstE):
    mesh = plsc.VectorSubcoreMesh(core_axis_name="c", subcore_axis_name="s")
    fn = functools.partial(
        pl.kernel,
        out_type=[
            jax.ShapeDtypeStruct((4 * N1, 128), F32),
            jax.ShapeDtypeStruct((2 * N1, 16), F32),
        ],
        mesh=mesh,
        scratch_types=[
            pltpu.VMEM((16,), F32),
            pltpu.VMEM((C1,), I32),
            pltpu.VMEM((C1,), I32),
            pltpu.VMEM((C1,), I32),
            pltpu.VMEM((C1, 16), F32),
            pltpu.VMEM((C1, 16), F32),
            pltpu.VMEM((C1, 16), F32),
            pltpu.VMEM((C1, 128), F32),
            pltpu.VMEM((32, 128), F32),
            pltpu.VMEM((32, 16), F32),
            pltpu.VMEM_SHARED((N1, 128), F32),
            pltpu.VMEM_SHARED((N1, 16), F32),
            pltpu.SemaphoreType.DMA,
            pltpu.SemaphoreType.DMA,
        ],
        compiler_params=_sc_compiler_params(),
    )(_sc1_body)
    return fn(tab1f, a_nodes, shift_rep, srcE, dstE)


def _tc_mid_body(msg_ref, den_ref, b1_ref, w2_ref, as2_ref, ad2_ref,
                 emb_ref, tab2_ref, a2n_ref, shift2_ref, mx_ref):
    i = pl.program_id(0)

    @pl.when(i == 0)
    def _():
        mx_ref[0] = -1e30
        mx_ref[1] = -1e30

    segs = []
    for h in range(H):
        seg = msg_ref[h // 2, :, (h % 2) * 64:(h % 2 + 1) * 64]
        dcol = den_ref[h // 4, :, (h % 4):(h % 4) + 1]
        segs.append(seg / (dcol + 1e-16))
    emb = jnp.concatenate(segs, axis=1) + b1_ref[...]
    emb_ref[...] = emb
    h2 = jnp.where(emb > 0, emb, jnp.exp(emb) - 1.0)
    h2m = lax.dot_general(h2, w2_ref[...], (((1,), (0,)), ((), ())),
                          precision=_HIGH, preferred_element_type=F32)
    tab2_ref[...] = h2m
    a_s = jnp.sum(h2m * as2_ref[...], axis=-1, keepdims=True)
    a_d = jnp.sum(h2m * ad2_ref[...], axis=-1, keepdims=True)
    col = lax.broadcasted_iota(I32, (a_s.shape[0], 16), 1)
    a2n_ref[...] = jnp.where(col == 0, a_s, jnp.where(col == 1, a_d, 0.0))
    mx_ref[0] = jnp.maximum(mx_ref[0], jnp.max(a_s))
    mx_ref[1] = jnp.maximum(mx_ref[1], jnp.max(a_d))
    shift2_ref[...] = jnp.full((8, 16), jnp.maximum(mx_ref[0] + mx_ref[1], 0.0), F32)


def _tc_mid(msg1, den1, b1_2d, W2, att_src2, att_dst2):
    BN = 256
    grid = (N1 // BN,)
    return pl.pallas_call(
        _tc_mid_body,
        grid=grid,
        in_specs=[
            pl.BlockSpec((4, BN, 128), lambda i: (0, i, 0)),
            pl.BlockSpec((2, BN, 16), lambda i: (0, i, 0)),
            pl.BlockSpec((1, H * HC), lambda i: (0, 0)),
            pl.BlockSpec((H * HC, NC), lambda i: (0, 0)),
            pl.BlockSpec((1, NC), lambda i: (0, 0)),
            pl.BlockSpec((1, NC), lambda i: (0, 0)),
        ],
        out_specs=[
            pl.BlockSpec((BN, H * HC), lambda i: (i, 0)),
            pl.BlockSpec((BN, NC), lambda i: (i, 0)),
            pl.BlockSpec((BN, 16), lambda i: (i, 0)),
            pl.BlockSpec((8, 16), lambda i: (0, 0)),
        ],
        out_shape=[
            jax.ShapeDtypeStruct((N1, H * HC), F32),
            jax.ShapeDtypeStruct((N1, NC), F32),
            jax.ShapeDtypeStruct((N1, 16), F32),
            jax.ShapeDtypeStruct((8, 16), F32),
        ],
        scratch_shapes=[pltpu.SMEM((2,), F32)],
    )(msg1, den1, b1_2d, W2, att_src2, att_dst2)


def _sc2_body(tab_hbm, an_hbm, shift_hbm, src_hbm, dst_hbm,
              msg_out, den_out,
              shift_v, src_v, dst_v, asg_v, adg_v, w16_v, msg_v, zbuf, zbuf16,
              acc_msg, acc_den, sem, sem2):
    c = lax.axis_index("c")
    s = lax.axis_index("s")
    epw = ET // 32
    nchunks = epw // C2

    pltpu.sync_copy(shift_hbm, shift_v)
    shift = shift_v[...]

    @pl.loop(0, 32)
    def _(i):
        for k in range(4):
            zbuf[i, pl.ds(k * 16, 16)] = jnp.zeros((16,), F32)
        zbuf16[i, pl.ds(0, 16)] = jnp.zeros((16,), F32)

    @pl.loop(0, C2)
    def _(i):
        w16_v[i, pl.ds(0, 16)] = jnp.zeros((16,), F32)

    r0 = s * RPW
    for k in range(RPW // 32):
        pltpu.sync_copy(zbuf, acc_msg.at[pl.ds(r0 + k * 32, 32)])
        pltpu.sync_copy(zbuf16, acc_den.at[pl.ds(r0 + k * 32, 32)])

    plsc.subcore_barrier()

    @pl.loop(0, nchunks)
    def _(t):
        base = c * (ET // 2) + s * epw + t * C2
        h1 = pltpu.async_copy(src_hbm.at[pl.ds(base, C2)], src_v, sem2)
        h2 = pltpu.async_copy(dst_hbm.at[pl.ds(base, C2)], dst_v, sem2)
        h1.wait()
        h2.wait()
        cp = pltpu.async_copy(tab_hbm.at[src_v], msg_v, sem)
        h3 = pltpu.async_copy(an_hbm.at[src_v], asg_v, sem2)
        h4 = pltpu.async_copy(an_hbm.at[dst_v], adg_v, sem2)
        h3.wait()
        h4.wait()

        @plsc.parallel_loop(0, C2, step=16, unroll=2)
        def _(g):
            rows = lax.iota(I32, 16) + g
            a0 = (plsc.load_gather(asg_v, [rows, jnp.full((16,), 0, I32)])
                  + plsc.load_gather(adg_v, [rows, jnp.full((16,), 1, I32)]))
            a0 = jnp.maximum(a0, 0.0) + 0.2 * jnp.minimum(a0, 0.0)
            w0 = jnp.exp(a0 - shift)
            plsc.store_scatter(w16_v, [rows, jnp.full((16,), 0, I32)], w0)

        cp.wait()

        @plsc.parallel_loop(0, C2, unroll=4)
        def _(i):
            wrow = w16_v[i, pl.ds(0, 16)]
            v0 = jnp.full((16,), wrow[0], F32)
            for k in range(4):
                msg_v[i, pl.ds(k * 16, 16)] = msg_v[i, pl.ds(k * 16, 16)] * v0

        h5 = pltpu.async_copy(msg_v, acc_msg.at[dst_v], sem2, add=True)
        h6 = pltpu.async_copy(w16_v, acc_den.at[dst_v], sem2, add=True)
        h5.wait()
        h6.wait()

    plsc.subcore_barrier()
    pltpu.sync_copy(acc_msg.at[pl.ds(r0, RPW)],
                    msg_out.at[pl.ds(c * N1 + r0, RPW)])
    pltpu.sync_copy(acc_den.at[pl.ds(r0, RPW)],
                    den_out.at[pl.ds(c * N1 + r0, RPW)])


def _sc_layer2(tab2, a2_nodes, shift2_rep, srcE, dstE):
    mesh = plsc.VectorSubcoreMesh(core_axis_name="c", subcore_axis_name="s")
    fn = functools.partial(
        pl.kernel,
        out_type=[
            jax.ShapeDtypeStruct((2 * N1, 64), F32),
            jax.ShapeDtypeStruct((2 * N1, 16), F32),
        ],
        mesh=mesh,
        scratch_types=[
            pltpu.VMEM((16,), F32),
            pltpu.VMEM((C2,), I32),
            pltpu.VMEM((C2,), I32),
            pltpu.VMEM((C2, 16), F32),
            pltpu.VMEM((C2, 16), F32),
            pltpu.VMEM((C2, 16), F32),
            pltpu.VMEM((C2, 64), F32),
            pltpu.VMEM((32, 64), F32),
            pltpu.VMEM((32, 16), F32),
            pltpu.VMEM_SHARED((N1, 64), F32),
            pltpu.VMEM_SHARED((N1, 16), F32),
            pltpu.SemaphoreType.DMA,
            pltpu.SemaphoreType.DMA,
        ],
        compiler_params=_sc_compiler_params(),
    )(_sc2_body)
    return fn(tab2, a2_nodes, shift2_rep, srcE, dstE)


def _tc_final_body(msg_ref, den_ref, b2_ref, out_ref):
    d = den_ref[0, :, 0:1] + den_ref[1, :, 0:1] + 1e-16
    out_ref[...] = (msg_ref[0] + msg_ref[1]) / d + b2_ref[...]


def _tc_final(msg2, den2, b2_2d):
    BN = 512
    grid = (N1 // BN,)
    return pl.pallas_call(
        _tc_final_body,
        grid=grid,
        in_specs=[
            pl.BlockSpec((2, BN, NC), lambda i: (0, i, 0)),
            pl.BlockSpec((2, BN, 16), lambda i: (0, i, 0)),
            pl.BlockSpec((1, NC), lambda i: (0, 0)),
        ],
        out_specs=pl.BlockSpec((BN, NC), lambda i: (i, 0)),
        out_shape=jax.ShapeDtypeStruct((N1, NC), F32),
    )(msg2, den2, b2_2d)


def kernel(x, edge_index, W1, att_src1, att_dst1, b1, W2, att_src2, att_dst2, b2):
    src = edge_index[0]
    dst = edge_index[1]
    loop = jnp.arange(N, dtype=src.dtype)
    fill = jnp.full((ET - E0 - N,), N, dtype=src.dtype)
    srcE = jnp.concatenate([src, loop, fill])
    dstE = jnp.concatenate([dst, loop, fill])

    x_p = jnp.pad(x, ((0, N1 - N), (0, 0)))

    tab1, a_nodes, shift = _tc_prep(x_p, W1, att_src1, att_dst1)
    tab1f = tab1.reshape(4 * N1, 128)
    shift_rep = shift[0]

    msg1, den1 = _sc_layer1(tab1f, a_nodes, shift_rep, srcE, dstE)
    msg1 = msg1.reshape(4, N1, 128)
    den1 = den1.reshape(2, N1, 16)

    emb_p, tab2, a2_nodes, shift2 = _tc_mid(msg1, den1, b1.reshape(1, H * HC),
                                            W2, att_src2, att_dst2)
    msg2, den2 = _sc_layer2(tab2, a2_nodes, shift2[0], srcE, dstE)
    logits_p = _tc_final(msg2.reshape(2, N1, 64), den2.reshape(2, N1, 16),
                         b2.reshape(1, NC))
    return (logits_p[:N], emb_p[:N])


# speedup vs baseline: 32.0181x; 1.0239x over previous
import dataclasses
import functools

import jax
import jax.numpy as jnp
from jax import lax
from jax.experimental import pallas as pl
from jax.experimental.pallas import tpu as pltpu
from jax.experimental.pallas import tpu_sc as plsc

N = 10000
NF = 128
HC = 64
H = 8
NC = 64
N1 = 10240
E0 = 320000
ET = 331776
RPW = N1 // 16
C1 = 128
C2 = 128

F32 = jnp.float32
I32 = jnp.int32

_HIGH = jax.lax.Precision.HIGHEST


def _sc_compiler_params():
    cp = pltpu.CompilerParams()
    fields = pltpu.CompilerParams.__dataclass_fields__
    if "needs_layout_passes" in fields:
        cp = dataclasses.replace(cp, needs_layout_passes=False)
    if "use_tc_tiling_on_sc" in fields:
        cp = dataclasses.replace(cp, use_tc_tiling_on_sc=False)
    return cp


def _tc_prep_body(x_ref, w1_ref, as_ref, ad_ref, tab_ref, an_ref, shift_ref, mx_ref):
    i = pl.program_id(0)

    @pl.when(i == 0)
    def _():
        mx_ref[0] = -1e30
        mx_ref[1] = -1e30

    h = lax.dot_general(x_ref[...], w1_ref[...], (((1,), (0,)), ((), ())),
                        precision=_HIGH, preferred_element_type=F32)
    hr = h.reshape(h.shape[0], H, HC)
    a_s = jnp.sum(hr * as_ref[...][None], axis=-1)
    a_d = jnp.sum(hr * ad_ref[...][None], axis=-1)
    for p in range(4):
        tab_ref[p] = h[:, p * 128:(p + 1) * 128]
    an_ref[...] = jnp.concatenate([a_s, a_d], axis=1)
    mx_ref[0] = jnp.maximum(mx_ref[0], jnp.max(a_s))
    mx_ref[1] = jnp.maximum(mx_ref[1], jnp.max(a_d))
    shift_ref[...] = jnp.full((8, 16), jnp.maximum(mx_ref[0] + mx_ref[1], 0.0), F32)


def _tc_prep(x_p, W1, att_src1, att_dst1):
    BN = 128
    grid = (N1 // BN,)
    return pl.pallas_call(
        _tc_prep_body,
        grid=grid,
        in_specs=[
            pl.BlockSpec((BN, NF), lambda i: (i, 0)),
            pl.BlockSpec((NF, H * HC), lambda i: (0, 0)),
            pl.BlockSpec((H, HC), lambda i: (0, 0)),
            pl.BlockSpec((H, HC), lambda i: (0, 0)),
        ],
        out_specs=[
            pl.BlockSpec((4, BN, 128), lambda i: (0, i, 0)),
            pl.BlockSpec((BN, 16), lambda i: (i, 0)),
            pl.BlockSpec((8, 16), lambda i: (0, 0)),
        ],
        out_shape=[
            jax.ShapeDtypeStruct((4, N1, 128), F32),
            jax.ShapeDtypeStruct((N1, 16), F32),
            jax.ShapeDtypeStruct((8, 16), F32),
        ],
        scratch_shapes=[pltpu.SMEM((2,), F32)],
    )(x_p, W1, att_src1, att_dst1)


def _sc1_body(tab_hbm, an_hbm, shift_hbm, src_hbm, dst_hbm,
              msg_out, den_out,
              shift_v, src_v0, dst_v0, idx_v0, asg_v0, adg_v0,
              src_v1, dst_v1, idx_v1, asg_v1, adg_v1,
              w16_v, msg_v, zbuf, zbuf16,
              acc_msg, acc_den, sem, sem2, semP, semA):
    c = lax.axis_index("c")
    s = lax.axis_index("s")
    srcs = (src_v0, src_v1)
    dsts = (dst_v0, dst_v1)
    idxs = (idx_v0, idx_v1)
    asgs = (asg_v0, asg_v1)
    adgs = (adg_v0, adg_v1)
    epw = ET // 16
    nchunks = epw // C1

    pltpu.sync_copy(shift_hbm, shift_v)
    shift = shift_v[...]

    @pl.loop(0, 32)
    def _(i):
        for k in range(8):
            zbuf[i, pl.ds(k * 16, 16)] = jnp.zeros((16,), F32)
        zbuf16[i, pl.ds(0, 16)] = jnp.zeros((16,), F32)

    r0 = s * RPW
    for p in range(2):
        P = 2 * c + p
        h0 = 4 * c + 2 * p

        for k in range(RPW // 32):
            pltpu.sync_copy(zbuf, acc_msg.at[pl.ds(r0 + k * 32, 32)])
            if p == 0:
                pltpu.sync_copy(zbuf16, acc_den.at[pl.ds(r0 + k * 32, 32)])

        @pl.loop(0, C1)
        def _(i):
            w16_v[i, pl.ds(0, 16)] = jnp.zeros((16,), F32)

        plsc.subcore_barrier()

        c0 = 2 * p

        e0 = s * epw
        hp1 = pltpu.async_copy(src_hbm.at[pl.ds(e0, C1)], srcs[0], sem2)
        hp2 = pltpu.async_copy(dst_hbm.at[pl.ds(e0, C1)], dsts[0], sem2)
        hp1.wait()
        hp2.wait()

        @plsc.parallel_loop(0, C1, step=16, unroll=2)
        def _(g):
            idxs[0][pl.ds(g, 16)] = srcs[0][pl.ds(g, 16)] + P * N1

        hp3 = pltpu.async_copy(an_hbm.at[srcs[0]], asgs[0], sem2)
        hp4 = pltpu.async_copy(an_hbm.at[dsts[0]], adgs[0], sem2)
        hp3.wait()
        hp4.wait()

        @pl.loop(0, nchunks, step=2)
        def _(tl):
            for b in range(2):
                ob = 1 - b
                t = tl + b
                src_v, dst_v, idx_v = srcs[b], dsts[b], idxs[b]
                asg_v, adg_v = asgs[b], adgs[b]

                cp = pltpu.async_copy(tab_hbm.at[idx_v], msg_v, sem)

                nbase = s * epw + (t + 1) * C1
                hp1 = pltpu.async_copy(src_hbm.at[pl.ds(nbase, C1)], srcs[ob], semP)
                hp2 = pltpu.async_copy(dst_hbm.at[pl.ds(nbase, C1)], dsts[ob], semP)

                @plsc.parallel_loop(0, C1, step=16, unroll=2)
                def _(g):
                    rows = lax.iota(I32, 16) + g
                    cs = jnp.full((16,), h0, I32)
                    cd = jnp.full((16,), 8 + h0, I32)
                    a0 = (plsc.load_gather(asg_v, [rows, cs])
                          + plsc.load_gather(adg_v, [rows, cd]))
                    a0 = jnp.maximum(a0, 0.0) + 0.2 * jnp.minimum(a0, 0.0)
                    w0 = jnp.exp(a0 - shift)
                    a1 = (plsc.load_gather(asg_v, [rows, cs + 1])
                          + plsc.load_gather(adg_v, [rows, cd + 1]))
                    a1 = jnp.maximum(a1, 0.0) + 0.2 * jnp.minimum(a1, 0.0)
                    w1 = jnp.exp(a1 - shift)
                    plsc.store_scatter(w16_v, [rows, jnp.full((16,), c0, I32)], w0)
                    plsc.store_scatter(w16_v, [rows, jnp.full((16,), c0 + 1, I32)], w1)

                cp.wait()

                @plsc.parallel_loop(0, C1, unroll=4)
                def _(i):
                    wrow = w16_v[i, pl.ds(0, 16)]
                    v0 = jnp.full((16,), wrow[c0], F32)
                    v1 = jnp.full((16,), wrow[c0 + 1], F32)
                    for k in range(4):
                        msg_v[i, pl.ds(k * 16, 16)] = msg_v[i, pl.ds(k * 16, 16)] * v0
                    for k in range(4, 8):
                        msg_v[i, pl.ds(k * 16, 16)] = msg_v[i, pl.ds(k * 16, 16)] * v1

                h5 = pltpu.async_copy(msg_v, acc_msg.at[dst_v], semA, add=True)
                h6 = pltpu.async_copy(w16_v, acc_den.at[dst_v], semA, add=True)

                hp1.wait()
                hp2.wait()

                @plsc.parallel_loop(0, C1, step=16, unroll=2)
                def _(g):
                    idxs[ob][pl.ds(g, 16)] = srcs[ob][pl.ds(g, 16)] + P * N1

                h7 = pltpu.async_copy(an_hbm.at[srcs[ob]], asgs[ob], sem2)
                h8 = pltpu.async_copy(an_hbm.at[dsts[ob]], adgs[ob], sem2)
                h7.wait()
                h8.wait()
                h5.wait()
                h6.wait()

        plsc.subcore_barrier()
        pltpu.sync_copy(acc_msg.at[pl.ds(r0, RPW)],
                        msg_out.at[pl.ds(P * N1 + r0, RPW)])
        if p == 1:
            pltpu.sync_copy(acc_den.at[pl.ds(r0, RPW)],
                            den_out.at[pl.ds(c * N1 + r0, RPW)])
        plsc.subcore_barrier()


def _sc_layer1(tab1f, a_nodes, shift_rep, srcE, dstE):
    mesh = plsc.VectorSubcoreMesh(core_axis_name="c", subcore_axis_name="s")
    fn = functools.partial(
        pl.kernel,
        out_type=[
            jax.ShapeDtypeStruct((4 * N1, 128), F32),
            jax.ShapeDtypeStruct((2 * N1, 16), F32),
        ],
        mesh=mesh,
        scratch_types=[
            pltpu.VMEM((16,), F32),
            pltpu.VMEM((C1,), I32),
            pltpu.VMEM((C1,), I32),
            pltpu.VMEM((C1,), I32),
            pltpu.VMEM((C1, 16), F32),
            pltpu.VMEM((C1, 16), F32),
            pltpu.VMEM((C1,), I32),
            pltpu.VMEM((C1,), I32),
            pltpu.VMEM((C1,), I32),
            pltpu.VMEM((C1, 16), F32),
            pltpu.VMEM((C1, 16), F32),
            pltpu.VMEM((C1, 16), F32),
            pltpu.VMEM((C1, 128), F32),
            pltpu.VMEM((32, 128), F32),
            pltpu.VMEM((32, 16), F32),
            pltpu.VMEM_SHARED((N1, 128), F32),
            pltpu.VMEM_SHARED((N1, 16), F32),
            pltpu.SemaphoreType.DMA,
            pltpu.SemaphoreType.DMA,
            pltpu.SemaphoreType.DMA,
            pltpu.SemaphoreType.DMA,
        ],
        compiler_params=_sc_compiler_params(),
    )(_sc1_body)
    return fn(tab1f, a_nodes, shift_rep, srcE, dstE)


def _tc_mid_body(msg_ref, den_ref, b1_ref, w2_ref, as2_ref, ad2_ref,
                 emb_ref, tab2_ref, a2n_ref, shift2_ref, mx_ref):
    i = pl.program_id(0)

    @pl.when(i == 0)
    def _():
        mx_ref[0] = -1e30
        mx_ref[1] = -1e30

    segs = []
    for h in range(H):
        seg = msg_ref[h // 2, :, (h % 2) * 64:(h % 2 + 1) * 64]
        dcol = den_ref[h // 4, :, (h % 4):(h % 4) + 1]
        segs.append(seg / (dcol + 1e-16))
    emb = jnp.concatenate(segs, axis=1) + b1_ref[...]
    emb_ref[...] = emb
    h2 = jnp.where(emb > 0, emb, jnp.exp(emb) - 1.0)
    h2m = lax.dot_general(h2, w2_ref[...], (((1,), (0,)), ((), ())),
                          precision=_HIGH, preferred_element_type=F32)
    tab2_ref[...] = h2m
    a_s = jnp.sum(h2m * as2_ref[...], axis=-1, keepdims=True)
    a_d = jnp.sum(h2m * ad2_ref[...], axis=-1, keepdims=True)
    col = lax.broadcasted_iota(I32, (a_s.shape[0], 16), 1)
    a2n_ref[...] = jnp.where(col == 0, a_s, jnp.where(col == 1, a_d, 0.0))
    mx_ref[0] = jnp.maximum(mx_ref[0], jnp.max(a_s))
    mx_ref[1] = jnp.maximum(mx_ref[1], jnp.max(a_d))
    shift2_ref[...] = jnp.full((8, 16), jnp.maximum(mx_ref[0] + mx_ref[1], 0.0), F32)


def _tc_mid(msg1, den1, b1_2d, W2, att_src2, att_dst2):
    BN = 256
    grid = (N1 // BN,)
    return pl.pallas_call(
        _tc_mid_body,
        grid=grid,
        in_specs=[
            pl.BlockSpec((4, BN, 128), lambda i: (0, i, 0)),
            pl.BlockSpec((2, BN, 16), lambda i: (0, i, 0)),
            pl.BlockSpec((1, H * HC), lambda i: (0, 0)),
            pl.BlockSpec((H * HC, NC), lambda i: (0, 0)),
            pl.BlockSpec((1, NC), lambda i: (0, 0)),
            pl.BlockSpec((1, NC), lambda i: (0, 0)),
        ],
        out_specs=[
            pl.BlockSpec((BN, H * HC), lambda i: (i, 0)),
            pl.BlockSpec((BN, NC), lambda i: (i, 0)),
            pl.BlockSpec((BN, 16), lambda i: (i, 0)),
            pl.BlockSpec((8, 16), lambda i: (0, 0)),
        ],
        out_shape=[
            jax.ShapeDtypeStruct((N1, H * HC), F32),
            jax.ShapeDtypeStruct((N1, NC), F32),
            jax.ShapeDtypeStruct((N1, 16), F32),
            jax.ShapeDtypeStruct((8, 16), F32),
        ],
        scratch_shapes=[pltpu.SMEM((2,), F32)],
    )(msg1, den1, b1_2d, W2, att_src2, att_dst2)


def _sc2_body(tab_hbm, an_hbm, shift_hbm, src_hbm, dst_hbm,
              msg_out, den_out,
              shift_v, src_v, dst_v, asg_v, adg_v, w16_v, msg_v, zbuf, zbuf16,
              acc_msg, acc_den, sem, sem2):
    c = lax.axis_index("c")
    s = lax.axis_index("s")
    epw = ET // 32
    nchunks = epw // C2

    pltpu.sync_copy(shift_hbm, shift_v)
    shift = shift_v[...]

    @pl.loop(0, 32)
    def _(i):
        for k in range(4):
            zbuf[i, pl.ds(k * 16, 16)] = jnp.zeros((16,), F32)
        zbuf16[i, pl.ds(0, 16)] = jnp.zeros((16,), F32)

    @pl.loop(0, C2)
    def _(i):
        w16_v[i, pl.ds(0, 16)] = jnp.zeros((16,), F32)

    r0 = s * RPW
    for k in range(RPW // 32):
        pltpu.sync_copy(zbuf, acc_msg.at[pl.ds(r0 + k * 32, 32)])
        pltpu.sync_copy(zbuf16, acc_den.at[pl.ds(r0 + k * 32, 32)])

    plsc.subcore_barrier()

    @pl.loop(0, nchunks)
    def _(t):
        base = c * (ET // 2) + s * epw + t * C2
        h1 = pltpu.async_copy(src_hbm.at[pl.ds(base, C2)], src_v, sem2)
        h2 = pltpu.async_copy(dst_hbm.at[pl.ds(base, C2)], dst_v, sem2)
        h1.wait()
        h2.wait()
        cp = pltpu.async_copy(tab_hbm.at[src_v], msg_v, sem)
        h3 = pltpu.async_copy(an_hbm.at[src_v], asg_v, sem2)
        h4 = pltpu.async_copy(an_hbm.at[dst_v], adg_v, sem2)
        h3.wait()
        h4.wait()

        @plsc.parallel_loop(0, C2, step=16, unroll=2)
        def _(g):
            rows = lax.iota(I32, 16) + g
            a0 = (plsc.load_gather(asg_v, [rows, jnp.full((16,), 0, I32)])
                  + plsc.load_gather(adg_v, [rows, jnp.full((16,), 1, I32)]))
            a0 = jnp.maximum(a0, 0.0) + 0.2 * jnp.minimum(a0, 0.0)
            w0 = jnp.exp(a0 - shift)
            plsc.store_scatter(w16_v, [rows, jnp.full((16,), 0, I32)], w0)

        cp.wait()

        @plsc.parallel_loop(0, C2, unroll=4)
        def _(i):
            wrow = w16_v[i, pl.ds(0, 16)]
            v0 = jnp.full((16,), wrow[0], F32)
            for k in range(4):
                msg_v[i, pl.ds(k * 16, 16)] = msg_v[i, pl.ds(k * 16, 16)] * v0

        h5 = pltpu.async_copy(msg_v, acc_msg.at[dst_v], sem2, add=True)
        h6 = pltpu.async_copy(w16_v, acc_den.at[dst_v], sem2, add=True)
        h5.wait()
        h6.wait()

    plsc.subcore_barrier()
    pltpu.sync_copy(acc_msg.at[pl.ds(r0, RPW)],
                    msg_out.at[pl.ds(c * N1 + r0, RPW)])
    pltpu.sync_copy(acc_den.at[pl.ds(r0, RPW)],
                    den_out.at[pl.ds(c * N1 + r0, RPW)])


def _sc_layer2(tab2, a2_nodes, shift2_rep, srcE, dstE):
    mesh = plsc.VectorSubcoreMesh(core_axis_name="c", subcore_axis_name="s")
    fn = functools.partial(
        pl.kernel,
        out_type=[
            jax.ShapeDtypeStruct((2 * N1, 64), F32),
            jax.ShapeDtypeStruct((2 * N1, 16), F32),
        ],
        mesh=mesh,
        scratch_types=[
            pltpu.VMEM((16,), F32),
            pltpu.VMEM((C2,), I32),
            pltpu.VMEM((C2,), I32),
            pltpu.VMEM((C2, 16), F32),
            pltpu.VMEM((C2, 16), F32),
            pltpu.VMEM((C2, 16), F32),
            pltpu.VMEM((C2, 64), F32),
            pltpu.VMEM((32, 64), F32),
            pltpu.VMEM((32, 16), F32),
            pltpu.VMEM_SHARED((N1, 64), F32),
            pltpu.VMEM_SHARED((N1, 16), F32),
            pltpu.SemaphoreType.DMA,
            pltpu.SemaphoreType.DMA,
        ],
        compiler_params=_sc_compiler_params(),
    )(_sc2_body)
    return fn(tab2, a2_nodes, shift2_rep, srcE, dstE)


def _tc_final_body(msg_ref, den_ref, b2_ref, out_ref):
    d = den_ref[0, :, 0:1] + den_ref[1, :, 0:1] + 1e-16
    out_ref[...] = (msg_ref[0] + msg_ref[1]) / d + b2_ref[...]


def _tc_final(msg2, den2, b2_2d):
    BN = 512
    grid = (N1 // BN,)
    return pl.pallas_call(
        _tc_final_body,
        grid=grid,
        in_specs=[
            pl.BlockSpec((2, BN, NC), lambda i: (0, i, 0)),
            pl.BlockSpec((2, BN, 16), lambda i: (0, i, 0)),
            pl.BlockSpec((1, NC), lambda i: (0, 0)),
        ],
        out_specs=pl.BlockSpec((BN, NC), lambda i: (i, 0)),
        out_shape=jax.ShapeDtypeStruct((N1, NC), F32),
    )(msg2, den2, b2_2d)


def kernel(x, edge_index, W1, att_src1, att_dst1, b1, W2, att_src2, att_dst2, b2):
    src = edge_index[0]
    dst = edge_index[1]
    loop = jnp.arange(N, dtype=src.dtype)
    fill = jnp.full((ET - E0 - N + 256,), N, dtype=src.dtype)
    srcE = jnp.concatenate([src, loop, fill])
    dstE = jnp.concatenate([dst, loop, fill])

    x_p = jnp.pad(x, ((0, N1 - N), (0, 0)))

    tab1, a_nodes, shift = _tc_prep(x_p, W1, att_src1, att_dst1)
    tab1f = tab1.reshape(4 * N1, 128)
    shift_rep = shift[0]

    msg1, den1 = _sc_layer1(tab1f, a_nodes, shift_rep, srcE, dstE)
    msg1 = msg1.reshape(4, N1, 128)
    den1 = den1.reshape(2, N1, 16)

    emb_p, tab2, a2_nodes, shift2 = _tc_mid(msg1, den1, b1.reshape(1, H * HC),
                                            W2, att_src2, att_dst2)
    msg2, den2 = _sc_layer2(tab2, a2_nodes, shift2[0], srcE, dstE)
    logits_p = _tc_final(msg2.reshape(2, N1, 64), den2.reshape(2, N1, 16),
                         b2.reshape(1, NC))
    return (logits_p[:N], emb_p[:N])


# speedup vs baseline: 35.4875x; 1.1084x over previous
import dataclasses
import functools

import jax
import jax.numpy as jnp
from jax import lax
from jax.experimental import pallas as pl
from jax.experimental.pallas import tpu as pltpu
from jax.experimental.pallas import tpu_sc as plsc

N = 10000
NF = 128
HC = 64
H = 8
NC = 64
N1 = 10240
E0 = 320000
ET = 331776
RPW = N1 // 16
C1 = 96
C2 = 128

F32 = jnp.float32
I32 = jnp.int32

_HIGH = jax.lax.Precision.HIGHEST


def _sc_compiler_params():
    cp = pltpu.CompilerParams()
    fields = pltpu.CompilerParams.__dataclass_fields__
    if "needs_layout_passes" in fields:
        cp = dataclasses.replace(cp, needs_layout_passes=False)
    if "use_tc_tiling_on_sc" in fields:
        cp = dataclasses.replace(cp, use_tc_tiling_on_sc=False)
    return cp


def _tc_prep_body(x_ref, w1_ref, as_ref, ad_ref, tab_ref, an_ref, shift_ref, mx_ref):
    i = pl.program_id(0)

    @pl.when(i == 0)
    def _():
        mx_ref[0] = -1e30
        mx_ref[1] = -1e30

    h = lax.dot_general(x_ref[...], w1_ref[...], (((1,), (0,)), ((), ())),
                        precision=_HIGH, preferred_element_type=F32)
    hr = h.reshape(h.shape[0], H, HC)
    a_s = jnp.sum(hr * as_ref[...][None], axis=-1)
    a_d = jnp.sum(hr * ad_ref[...][None], axis=-1)
    for p in range(4):
        tab_ref[p] = h[:, p * 128:(p + 1) * 128]
    an_ref[...] = jnp.concatenate([a_s, a_d], axis=1)
    mx_ref[0] = jnp.maximum(mx_ref[0], jnp.max(a_s))
    mx_ref[1] = jnp.maximum(mx_ref[1], jnp.max(a_d))
    shift_ref[...] = jnp.full((8, 16), jnp.maximum(mx_ref[0] + mx_ref[1], 0.0), F32)


def _tc_prep(x_p, W1, att_src1, att_dst1):
    BN = 128
    grid = (N1 // BN,)
    return pl.pallas_call(
        _tc_prep_body,
        grid=grid,
        in_specs=[
            pl.BlockSpec((BN, NF), lambda i: (i, 0)),
            pl.BlockSpec((NF, H * HC), lambda i: (0, 0)),
            pl.BlockSpec((H, HC), lambda i: (0, 0)),
            pl.BlockSpec((H, HC), lambda i: (0, 0)),
        ],
        out_specs=[
            pl.BlockSpec((4, BN, 128), lambda i: (0, i, 0)),
            pl.BlockSpec((BN, 16), lambda i: (i, 0)),
            pl.BlockSpec((8, 16), lambda i: (0, 0)),
        ],
        out_shape=[
            jax.ShapeDtypeStruct((4, N1, 128), F32),
            jax.ShapeDtypeStruct((N1, 16), F32),
            jax.ShapeDtypeStruct((8, 16), F32),
        ],
        scratch_shapes=[pltpu.SMEM((2,), F32)],
    )(x_p, W1, att_src1, att_dst1)


def _sc1_body(tab_hbm, an_hbm, shift_hbm, src_hbm, dst_hbm,
              msg_out, den_out,
              shift_v, src_v0, dst_v0, idx_v0, asg_v0, adg_v0, w16_v0, msg_v0,
              src_v1, dst_v1, idx_v1, asg_v1, adg_v1, w16_v1, msg_v1,
              zbuf, zbuf16,
              acc_msg, acc_den, sem, sem2, semP, semA):
    c = lax.axis_index("c")
    s = lax.axis_index("s")
    srcs = (src_v0, src_v1)
    dsts = (dst_v0, dst_v1)
    idxs = (idx_v0, idx_v1)
    asgs = (asg_v0, asg_v1)
    adgs = (adg_v0, adg_v1)
    w16s = (w16_v0, w16_v1)
    msgs = (msg_v0, msg_v1)
    epw = ET // 16
    nchunks = epw // C1

    pltpu.sync_copy(shift_hbm, shift_v)
    shift = shift_v[...]

    @pl.loop(0, 16)
    def _(i):
        for k in range(8):
            zbuf[i, pl.ds(k * 16, 16)] = jnp.zeros((16,), F32)
        zbuf16[i, pl.ds(0, 16)] = jnp.zeros((16,), F32)

    r0 = s * RPW
    for p in range(2):
        P = 2 * c + p
        h0 = 4 * c + 2 * p

        for k in range(RPW // 16):
            pltpu.sync_copy(zbuf, acc_msg.at[pl.ds(r0 + k * 16, 16)])
            if p == 0:
                pltpu.sync_copy(zbuf16, acc_den.at[pl.ds(r0 + k * 16, 16)])

        for b in range(2):
            @pl.loop(0, C1)
            def _(i):
                w16s[b][i, pl.ds(0, 16)] = jnp.zeros((16,), F32)

        plsc.subcore_barrier()

        c0 = 2 * p

        e0 = s * epw
        hp1 = pltpu.async_copy(src_hbm.at[pl.ds(e0, C1)], srcs[0], semP)
        hp2 = pltpu.async_copy(dst_hbm.at[pl.ds(e0, C1)], dsts[0], semP)
        hp1.wait()
        hp2.wait()

        @plsc.parallel_loop(0, C1, step=16, unroll=2)
        def _(g):
            idxs[0][pl.ds(g, 16)] = srcs[0][pl.ds(g, 16)] + P * N1

        pltpu.async_copy(tab_hbm.at[idxs[0]], msgs[0], sem)
        hp3 = pltpu.async_copy(an_hbm.at[srcs[0]], asgs[0], sem2)
        hp4 = pltpu.async_copy(an_hbm.at[dsts[0]], adgs[0], sem2)
        hp3.wait()
        hp4.wait()

        @pl.loop(0, nchunks, step=2)
        def _(tl):
            for b in range(2):
                ob = 1 - b
                t = tl + b
                src_v, dst_v, idx_v = srcs[b], dsts[b], idxs[b]
                asg_v, adg_v = asgs[b], adgs[b]
                w16_v, msg_v = w16s[b], msgs[b]

                nbase = s * epw + (t + 1) * C1
                hp1 = pltpu.async_copy(src_hbm.at[pl.ds(nbase, C1)], srcs[ob], semP)
                hp2 = pltpu.async_copy(dst_hbm.at[pl.ds(nbase, C1)], dsts[ob], semP)

                @plsc.parallel_loop(0, C1, step=16, unroll=2)
                def _(g):
                    rows = lax.iota(I32, 16) + g
                    cs = jnp.full((16,), h0, I32)
                    cd = jnp.full((16,), 8 + h0, I32)
                    a0 = (plsc.load_gather(asg_v, [rows, cs])
                          + plsc.load_gather(adg_v, [rows, cd]))
                    a0 = jnp.maximum(a0, 0.0) + 0.2 * jnp.minimum(a0, 0.0)
                    w0 = jnp.exp(a0 - shift)
                    a1 = (plsc.load_gather(asg_v, [rows, cs + 1])
                          + plsc.load_gather(adg_v, [rows, cd + 1]))
                    a1 = jnp.maximum(a1, 0.0) + 0.2 * jnp.minimum(a1, 0.0)
                    w1 = jnp.exp(a1 - shift)
                    plsc.store_scatter(w16_v, [rows, jnp.full((16,), c0, I32)], w0)
                    plsc.store_scatter(w16_v, [rows, jnp.full((16,), c0 + 1, I32)], w1)

                pltpu.make_async_copy(tab_hbm.at[idx_v], msg_v, sem).wait()

                @plsc.parallel_loop(0, C1, unroll=4)
                def _(i):
                    wrow = w16_v[i, pl.ds(0, 16)]
                    v0 = jnp.full((16,), wrow[c0], F32)
                    v1 = jnp.full((16,), wrow[c0 + 1], F32)
                    for k in range(4):
                        msg_v[i, pl.ds(k * 16, 16)] = msg_v[i, pl.ds(k * 16, 16)] * v0
                    for k in range(4, 8):
                        msg_v[i, pl.ds(k * 16, 16)] = msg_v[i, pl.ds(k * 16, 16)] * v1

                h5 = pltpu.async_copy(msg_v, acc_msg.at[dst_v], semA, add=True)
                h6 = pltpu.async_copy(w16_v, acc_den.at[dst_v], semA, add=True)

                hp1.wait()
                hp2.wait()

                @plsc.parallel_loop(0, C1, step=16, unroll=2)
                def _(g):
                    idxs[ob][pl.ds(g, 16)] = srcs[ob][pl.ds(g, 16)] + P * N1

                pltpu.async_copy(tab_hbm.at[idxs[ob]], msgs[ob], sem)
                h7 = pltpu.async_copy(an_hbm.at[srcs[ob]], asgs[ob], sem2)
                h8 = pltpu.async_copy(an_hbm.at[dsts[ob]], adgs[ob], sem2)
                h7.wait()
                h8.wait()
                h5.wait()
                h6.wait()

        pltpu.make_async_copy(tab_hbm.at[idxs[0]], msgs[0], sem).wait()

        plsc.subcore_barrier()
        pltpu.sync_copy(acc_msg.at[pl.ds(r0, RPW)],
                        msg_out.at[pl.ds(P * N1 + r0, RPW)])
        if p == 1:
            pltpu.sync_copy(acc_den.at[pl.ds(r0, RPW)],
                            den_out.at[pl.ds(c * N1 + r0, RPW)])
        plsc.subcore_barrier()


def _sc_layer1(tab1f, a_nodes, shift_rep, srcE, dstE):
    mesh = plsc.VectorSubcoreMesh(core_axis_name="c", subcore_axis_name="s")
    fn = functools.partial(
        pl.kernel,
        out_type=[
            jax.ShapeDtypeStruct((4 * N1, 128), F32),
            jax.ShapeDtypeStruct((2 * N1, 16), F32),
        ],
        mesh=mesh,
        scratch_types=[
            pltpu.VMEM((16,), F32),
            pltpu.VMEM((C1,), I32),
            pltpu.VMEM((C1,), I32),
            pltpu.VMEM((C1,), I32),
            pltpu.VMEM((C1, 16), F32),
            pltpu.VMEM((C1, 16), F32),
            pltpu.VMEM((C1, 16), F32),
            pltpu.VMEM((C1, 128), F32),
            pltpu.VMEM((C1,), I32),
            pltpu.VMEM((C1,), I32),
            pltpu.VMEM((C1,), I32),
            pltpu.VMEM((C1, 16), F32),
            pltpu.VMEM((C1, 16), F32),
            pltpu.VMEM((C1, 16), F32),
            pltpu.VMEM((C1, 128), F32),
            pltpu.VMEM((16, 128), F32),
            pltpu.VMEM((16, 16), F32),
            pltpu.VMEM_SHARED((N1, 128), F32),
            pltpu.VMEM_SHARED((N1, 16), F32),
            pltpu.SemaphoreType.DMA,
            pltpu.SemaphoreType.DMA,
            pltpu.SemaphoreType.DMA,
            pltpu.SemaphoreType.DMA,
        ],
        compiler_params=_sc_compiler_params(),
    )(_sc1_body)
    return fn(tab1f, a_nodes, shift_rep, srcE, dstE)


def _tc_mid_body(msg_ref, den_ref, b1_ref, w2_ref, as2_ref, ad2_ref,
                 emb_ref, tab2_ref, a2n_ref, shift2_ref, mx_ref):
    i = pl.program_id(0)

    @pl.when(i == 0)
    def _():
        mx_ref[0] = -1e30
        mx_ref[1] = -1e30

    segs = []
    for h in range(H):
        seg = msg_ref[h // 2, :, (h % 2) * 64:(h % 2 + 1) * 64]
        dcol = den_ref[h // 4, :, (h % 4):(h % 4) + 1]
        segs.append(seg / (dcol + 1e-16))
    emb = jnp.concatenate(segs, axis=1) + b1_ref[...]
    emb_ref[...] = emb
    h2 = jnp.where(emb > 0, emb, jnp.exp(emb) - 1.0)
    h2m = lax.dot_general(h2, w2_ref[...], (((1,), (0,)), ((), ())),
                          precision=_HIGH, preferred_element_type=F32)
    tab2_ref[...] = h2m
    a_s = jnp.sum(h2m * as2_ref[...], axis=-1, keepdims=True)
    a_d = jnp.sum(h2m * ad2_ref[...], axis=-1, keepdims=True)
    col = lax.broadcasted_iota(I32, (a_s.shape[0], 16), 1)
    a2n_ref[...] = jnp.where(col == 0, a_s, jnp.where(col == 1, a_d, 0.0))
    mx_ref[0] = jnp.maximum(mx_ref[0], jnp.max(a_s))
    mx_ref[1] = jnp.maximum(mx_ref[1], jnp.max(a_d))
    shift2_ref[...] = jnp.full((8, 16), jnp.maximum(mx_ref[0] + mx_ref[1], 0.0), F32)


def _tc_mid(msg1, den1, b1_2d, W2, att_src2, att_dst2):
    BN = 256
    grid = (N1 // BN,)
    return pl.pallas_call(
        _tc_mid_body,
        grid=grid,
        in_specs=[
            pl.BlockSpec((4, BN, 128), lambda i: (0, i, 0)),
            pl.BlockSpec((2, BN, 16), lambda i: (0, i, 0)),
            pl.BlockSpec((1, H * HC), lambda i: (0, 0)),
            pl.BlockSpec((H * HC, NC), lambda i: (0, 0)),
            pl.BlockSpec((1, NC), lambda i: (0, 0)),
            pl.BlockSpec((1, NC), lambda i: (0, 0)),
        ],
        out_specs=[
            pl.BlockSpec((BN, H * HC), lambda i: (i, 0)),
            pl.BlockSpec((BN, NC), lambda i: (i, 0)),
            pl.BlockSpec((BN, 16), lambda i: (i, 0)),
            pl.BlockSpec((8, 16), lambda i: (0, 0)),
        ],
        out_shape=[
            jax.ShapeDtypeStruct((N1, H * HC), F32),
            jax.ShapeDtypeStruct((N1, NC), F32),
            jax.ShapeDtypeStruct((N1, 16), F32),
            jax.ShapeDtypeStruct((8, 16), F32),
        ],
        scratch_shapes=[pltpu.SMEM((2,), F32)],
    )(msg1, den1, b1_2d, W2, att_src2, att_dst2)


def _sc2_body(tab_hbm, an_hbm, shift_hbm, src_hbm, dst_hbm,
              msg_out, den_out,
              shift_v, src_v, dst_v, asg_v, adg_v, w16_v, msg_v, zbuf, zbuf16,
              acc_msg, acc_den, sem, sem2):
    c = lax.axis_index("c")
    s = lax.axis_index("s")
    epw = ET // 32
    nchunks = epw // C2

    pltpu.sync_copy(shift_hbm, shift_v)
    shift = shift_v[...]

    @pl.loop(0, 32)
    def _(i):
        for k in range(4):
            zbuf[i, pl.ds(k * 16, 16)] = jnp.zeros((16,), F32)
        zbuf16[i, pl.ds(0, 16)] = jnp.zeros((16,), F32)

    @pl.loop(0, C2)
    def _(i):
        w16_v[i, pl.ds(0, 16)] = jnp.zeros((16,), F32)

    r0 = s * RPW
    for k in range(RPW // 32):
        pltpu.sync_copy(zbuf, acc_msg.at[pl.ds(r0 + k * 32, 32)])
        pltpu.sync_copy(zbuf16, acc_den.at[pl.ds(r0 + k * 32, 32)])

    plsc.subcore_barrier()

    @pl.loop(0, nchunks)
    def _(t):
        base = c * (ET // 2) + s * epw + t * C2
        h1 = pltpu.async_copy(src_hbm.at[pl.ds(base, C2)], src_v, sem2)
        h2 = pltpu.async_copy(dst_hbm.at[pl.ds(base, C2)], dst_v, sem2)
        h1.wait()
        h2.wait()
        cp = pltpu.async_copy(tab_hbm.at[src_v], msg_v, sem)
        h3 = pltpu.async_copy(an_hbm.at[src_v], asg_v, sem2)
        h4 = pltpu.async_copy(an_hbm.at[dst_v], adg_v, sem2)
        h3.wait()
        h4.wait()

        @plsc.parallel_loop(0, C2, step=16, unroll=2)
        def _(g):
            rows = lax.iota(I32, 16) + g
            a0 = (plsc.load_gather(asg_v, [rows, jnp.full((16,), 0, I32)])
                  + plsc.load_gather(adg_v, [rows, jnp.full((16,), 1, I32)]))
            a0 = jnp.maximum(a0, 0.0) + 0.2 * jnp.minimum(a0, 0.0)
            w0 = jnp.exp(a0 - shift)
            plsc.store_scatter(w16_v, [rows, jnp.full((16,), 0, I32)], w0)

        cp.wait()

        @plsc.parallel_loop(0, C2, unroll=4)
        def _(i):
            wrow = w16_v[i, pl.ds(0, 16)]
            v0 = jnp.full((16,), wrow[0], F32)
            for k in range(4):
                msg_v[i, pl.ds(k * 16, 16)] = msg_v[i, pl.ds(k * 16, 16)] * v0

        h5 = pltpu.async_copy(msg_v, acc_msg.at[dst_v], sem2, add=True)
        h6 = pltpu.async_copy(w16_v, acc_den.at[dst_v], sem2, add=True)
        h5.wait()
        h6.wait()

    plsc.subcore_barrier()
    pltpu.sync_copy(acc_msg.at[pl.ds(r0, RPW)],
                    msg_out.at[pl.ds(c * N1 + r0, RPW)])
    pltpu.sync_copy(acc_den.at[pl.ds(r0, RPW)],
                    den_out.at[pl.ds(c * N1 + r0, RPW)])


def _sc_layer2(tab2, a2_nodes, shift2_rep, srcE, dstE):
    mesh = plsc.VectorSubcoreMesh(core_axis_name="c", subcore_axis_name="s")
    fn = functools.partial(
        pl.kernel,
        out_type=[
            jax.ShapeDtypeStruct((2 * N1, 64), F32),
            jax.ShapeDtypeStruct((2 * N1, 16), F32),
        ],
        mesh=mesh,
        scratch_types=[
            pltpu.VMEM((16,), F32),
            pltpu.VMEM((C2,), I32),
            pltpu.VMEM((C2,), I32),
            pltpu.VMEM((C2, 16), F32),
            pltpu.VMEM((C2, 16), F32),
            pltpu.VMEM((C2, 16), F32),
            pltpu.VMEM((C2, 64), F32),
            pltpu.VMEM((32, 64), F32),
            pltpu.VMEM((32, 16), F32),
            pltpu.VMEM_SHARED((N1, 64), F32),
            pltpu.VMEM_SHARED((N1, 16), F32),
            pltpu.SemaphoreType.DMA,
            pltpu.SemaphoreType.DMA,
        ],
        compiler_params=_sc_compiler_params(),
    )(_sc2_body)
    return fn(tab2, a2_nodes, shift2_rep, srcE, dstE)


def _tc_final_body(msg_ref, den_ref, b2_ref, out_ref):
    d = den_ref[0, :, 0:1] + den_ref[1, :, 0:1] + 1e-16
    out_ref[...] = (msg_ref[0] + msg_ref[1]) / d + b2_ref[...]


def _tc_final(msg2, den2, b2_2d):
    BN = 512
    grid = (N1 // BN,)
    return pl.pallas_call(
        _tc_final_body,
        grid=grid,
        in_specs=[
            pl.BlockSpec((2, BN, NC), lambda i: (0, i, 0)),
            pl.BlockSpec((2, BN, 16), lambda i: (0, i, 0)),
            pl.BlockSpec((1, NC), lambda i: (0, 0)),
        ],
        out_specs=pl.BlockSpec((BN, NC), lambda i: (i, 0)),
        out_shape=jax.ShapeDtypeStruct((N1, NC), F32),
    )(msg2, den2, b2_2d)


def kernel(x, edge_index, W1, att_src1, att_dst1, b1, W2, att_src2, att_dst2, b2):
    src = edge_index[0]
    dst = edge_index[1]
    loop = jnp.arange(N, dtype=src.dtype)
    fill = jnp.full((ET - E0 - N + 256,), N, dtype=src.dtype)
    srcE = jnp.concatenate([src, loop, fill])
    dstE = jnp.concatenate([dst, loop, fill])

    x_p = jnp.pad(x, ((0, N1 - N), (0, 0)))

    tab1, a_nodes, shift = _tc_prep(x_p, W1, att_src1, att_dst1)
    tab1f = tab1.reshape(4 * N1, 128)
    shift_rep = shift[0]

    msg1, den1 = _sc_layer1(tab1f, a_nodes, shift_rep, srcE, dstE)
    msg1 = msg1.reshape(4, N1, 128)
    den1 = den1.reshape(2, N1, 16)

    emb_p, tab2, a2_nodes, shift2 = _tc_mid(msg1, den1, b1.reshape(1, H * HC),
                                            W2, att_src2, att_dst2)
    msg2, den2 = _sc_layer2(tab2, a2_nodes, shift2[0], srcE, dstE)
    logits_p = _tc_final(msg2.reshape(2, N1, 64), den2.reshape(2, N1, 16),
                         b2.reshape(1, NC))
    return (logits_p[:N], emb_p[:N])


# speedup vs baseline: 36.7485x; 1.0355x over previous
import dataclasses
import functools

import jax
import jax.numpy as jnp
from jax import lax
from jax.experimental import pallas as pl
from jax.experimental.pallas import tpu as pltpu
from jax.experimental.pallas import tpu_sc as plsc

N = 10000
NF = 128
HC = 64
H = 8
NC = 64
N1 = 10240
E0 = 320000
ET = 331776
RPW = N1 // 16
C1 = 96
C2 = 96

F32 = jnp.float32
I32 = jnp.int32

_HIGH = jax.lax.Precision.HIGHEST


def _sc_compiler_params():
    cp = pltpu.CompilerParams()
    fields = pltpu.CompilerParams.__dataclass_fields__
    if "needs_layout_passes" in fields:
        cp = dataclasses.replace(cp, needs_layout_passes=False)
    if "use_tc_tiling_on_sc" in fields:
        cp = dataclasses.replace(cp, use_tc_tiling_on_sc=False)
    return cp


def _tc_prep_body(x_ref, w1_ref, as_ref, ad_ref, tab_ref, an_ref, shift_ref, mx_ref):
    i = pl.program_id(0)

    @pl.when(i == 0)
    def _():
        mx_ref[0] = -1e30
        mx_ref[1] = -1e30

    h = lax.dot_general(x_ref[...], w1_ref[...], (((1,), (0,)), ((), ())),
                        precision=_HIGH, preferred_element_type=F32)
    hr = h.reshape(h.shape[0], H, HC)
    a_s = jnp.sum(hr * as_ref[...][None], axis=-1)
    a_d = jnp.sum(hr * ad_ref[...][None], axis=-1)
    for p in range(4):
        tab_ref[p] = h[:, p * 128:(p + 1) * 128]
    an_ref[...] = jnp.concatenate([a_s, a_d], axis=1)
    mx_ref[0] = jnp.maximum(mx_ref[0], jnp.max(a_s))
    mx_ref[1] = jnp.maximum(mx_ref[1], jnp.max(a_d))
    shift_ref[...] = jnp.full((8, 16), jnp.maximum(mx_ref[0] + mx_ref[1], 0.0), F32)


def _tc_prep(x_p, W1, att_src1, att_dst1):
    BN = 128
    grid = (N1 // BN,)
    return pl.pallas_call(
        _tc_prep_body,
        grid=grid,
        in_specs=[
            pl.BlockSpec((BN, NF), lambda i: (i, 0)),
            pl.BlockSpec((NF, H * HC), lambda i: (0, 0)),
            pl.BlockSpec((H, HC), lambda i: (0, 0)),
            pl.BlockSpec((H, HC), lambda i: (0, 0)),
        ],
        out_specs=[
            pl.BlockSpec((4, BN, 128), lambda i: (0, i, 0)),
            pl.BlockSpec((BN, 16), lambda i: (i, 0)),
            pl.BlockSpec((8, 16), lambda i: (0, 0)),
        ],
        out_shape=[
            jax.ShapeDtypeStruct((4, N1, 128), F32),
            jax.ShapeDtypeStruct((N1, 16), F32),
            jax.ShapeDtypeStruct((8, 16), F32),
        ],
        scratch_shapes=[pltpu.SMEM((2,), F32)],
    )(x_p, W1, att_src1, att_dst1)


def _sc1_body(tab_hbm, an_hbm, shift_hbm, src_hbm, dst_hbm,
              msg_out, den_out,
              shift_v, src_v0, dst_v0, idx_v0, asg_v0, adg_v0, w16_v0, msg_v0,
              src_v1, dst_v1, idx_v1, asg_v1, adg_v1, w16_v1, msg_v1,
              zbuf, zbuf16,
              acc_msg, acc_den, sem, sem2, semP, semA):
    c = lax.axis_index("c")
    s = lax.axis_index("s")
    srcs = (src_v0, src_v1)
    dsts = (dst_v0, dst_v1)
    idxs = (idx_v0, idx_v1)
    asgs = (asg_v0, asg_v1)
    adgs = (adg_v0, adg_v1)
    w16s = (w16_v0, w16_v1)
    msgs = (msg_v0, msg_v1)
    epw = ET // 16
    nchunks = epw // C1

    pltpu.sync_copy(shift_hbm, shift_v)
    shift = shift_v[...]

    @pl.loop(0, 16)
    def _(i):
        for k in range(8):
            zbuf[i, pl.ds(k * 16, 16)] = jnp.zeros((16,), F32)
        zbuf16[i, pl.ds(0, 16)] = jnp.zeros((16,), F32)

    r0 = s * RPW
    for p in range(2):
        P = 2 * c + p
        h0 = 4 * c + 2 * p

        for k in range(RPW // 16):
            pltpu.sync_copy(zbuf, acc_msg.at[pl.ds(r0 + k * 16, 16)])
            if p == 0:
                pltpu.sync_copy(zbuf16, acc_den.at[pl.ds(r0 + k * 16, 16)])

        for b in range(2):
            @pl.loop(0, C1)
            def _(i):
                w16s[b][i, pl.ds(0, 16)] = jnp.zeros((16,), F32)

        plsc.subcore_barrier()

        c0 = 2 * p

        e0 = s * epw
        hp1 = pltpu.async_copy(src_hbm.at[pl.ds(e0, C1)], srcs[0], semP)
        hp2 = pltpu.async_copy(dst_hbm.at[pl.ds(e0, C1)], dsts[0], semP)
        hp1.wait()
        hp2.wait()

        @plsc.parallel_loop(0, C1, step=16, unroll=2)
        def _(g):
            idxs[0][pl.ds(g, 16)] = srcs[0][pl.ds(g, 16)] + P * N1

        pltpu.async_copy(tab_hbm.at[idxs[0]], msgs[0], sem)
        hp3 = pltpu.async_copy(an_hbm.at[srcs[0]], asgs[0], sem2)
        hp4 = pltpu.async_copy(an_hbm.at[dsts[0]], adgs[0], sem2)
        hp3.wait()
        hp4.wait()

        @pl.loop(0, nchunks, step=2)
        def _(tl):
            for b in range(2):
                ob = 1 - b
                t = tl + b
                src_v, dst_v, idx_v = srcs[b], dsts[b], idxs[b]
                asg_v, adg_v = asgs[b], adgs[b]
                w16_v, msg_v = w16s[b], msgs[b]

                nbase = s * epw + (t + 1) * C1
                hp1 = pltpu.async_copy(src_hbm.at[pl.ds(nbase, C1)], srcs[ob], semP)
                hp2 = pltpu.async_copy(dst_hbm.at[pl.ds(nbase, C1)], dsts[ob], semP)

                @plsc.parallel_loop(0, C1, step=16, unroll=2)
                def _(g):
                    rows = lax.iota(I32, 16) + g
                    cs = jnp.full((16,), h0, I32)
                    cd = jnp.full((16,), 8 + h0, I32)
                    a0 = (plsc.load_gather(asg_v, [rows, cs])
                          + plsc.load_gather(adg_v, [rows, cd]))
                    a0 = jnp.maximum(a0, 0.0) + 0.2 * jnp.minimum(a0, 0.0)
                    w0 = jnp.exp(a0 - shift)
                    a1 = (plsc.load_gather(asg_v, [rows, cs + 1])
                          + plsc.load_gather(adg_v, [rows, cd + 1]))
                    a1 = jnp.maximum(a1, 0.0) + 0.2 * jnp.minimum(a1, 0.0)
                    w1 = jnp.exp(a1 - shift)
                    plsc.store_scatter(w16_v, [rows, jnp.full((16,), c0, I32)], w0)
                    plsc.store_scatter(w16_v, [rows, jnp.full((16,), c0 + 1, I32)], w1)

                pltpu.make_async_copy(tab_hbm.at[idx_v], msg_v, sem).wait()

                @plsc.parallel_loop(0, C1, unroll=4)
                def _(i):
                    wrow = w16_v[i, pl.ds(0, 16)]
                    v0 = jnp.full((16,), wrow[c0], F32)
                    v1 = jnp.full((16,), wrow[c0 + 1], F32)
                    for k in range(4):
                        msg_v[i, pl.ds(k * 16, 16)] = msg_v[i, pl.ds(k * 16, 16)] * v0
                    for k in range(4, 8):
                        msg_v[i, pl.ds(k * 16, 16)] = msg_v[i, pl.ds(k * 16, 16)] * v1

                h5 = pltpu.async_copy(msg_v, acc_msg.at[dst_v], semA, add=True)
                h6 = pltpu.async_copy(w16_v, acc_den.at[dst_v], semA, add=True)

                hp1.wait()
                hp2.wait()

                @plsc.parallel_loop(0, C1, step=16, unroll=2)
                def _(g):
                    idxs[ob][pl.ds(g, 16)] = srcs[ob][pl.ds(g, 16)] + P * N1

                pltpu.async_copy(tab_hbm.at[idxs[ob]], msgs[ob], sem)
                h7 = pltpu.async_copy(an_hbm.at[srcs[ob]], asgs[ob], sem2)
                h8 = pltpu.async_copy(an_hbm.at[dsts[ob]], adgs[ob], sem2)
                h7.wait()
                h8.wait()
                h5.wait()
                h6.wait()

        pltpu.make_async_copy(tab_hbm.at[idxs[0]], msgs[0], sem).wait()

        plsc.subcore_barrier()
        pltpu.sync_copy(acc_msg.at[pl.ds(r0, RPW)],
                        msg_out.at[pl.ds(P * N1 + r0, RPW)])
        if p == 1:
            pltpu.sync_copy(acc_den.at[pl.ds(r0, RPW)],
                            den_out.at[pl.ds(c * N1 + r0, RPW)])
        plsc.subcore_barrier()


def _sc_layer1(tab1f, a_nodes, shift_rep, srcE, dstE):
    mesh = plsc.VectorSubcoreMesh(core_axis_name="c", subcore_axis_name="s")
    fn = functools.partial(
        pl.kernel,
        out_type=[
            jax.ShapeDtypeStruct((4 * N1, 128), F32),
            jax.ShapeDtypeStruct((2 * N1, 16), F32),
        ],
        mesh=mesh,
        scratch_types=[
            pltpu.VMEM((16,), F32),
            pltpu.VMEM((C1,), I32),
            pltpu.VMEM((C1,), I32),
            pltpu.VMEM((C1,), I32),
            pltpu.VMEM((C1, 16), F32),
            pltpu.VMEM((C1, 16), F32),
            pltpu.VMEM((C1, 16), F32),
            pltpu.VMEM((C1, 128), F32),
            pltpu.VMEM((C1,), I32),
            pltpu.VMEM((C1,), I32),
            pltpu.VMEM((C1,), I32),
            pltpu.VMEM((C1, 16), F32),
            pltpu.VMEM((C1, 16), F32),
            pltpu.VMEM((C1, 16), F32),
            pltpu.VMEM((C1, 128), F32),
            pltpu.VMEM((16, 128), F32),
            pltpu.VMEM((16, 16), F32),
            pltpu.VMEM_SHARED((N1, 128), F32),
            pltpu.VMEM_SHARED((N1, 16), F32),
            pltpu.SemaphoreType.DMA,
            pltpu.SemaphoreType.DMA,
            pltpu.SemaphoreType.DMA,
            pltpu.SemaphoreType.DMA,
        ],
        compiler_params=_sc_compiler_params(),
    )(_sc1_body)
    return fn(tab1f, a_nodes, shift_rep, srcE, dstE)


def _tc_mid_body(msg_ref, den_ref, b1_ref, w2_ref, as2_ref, ad2_ref,
                 emb_ref, tab2_ref, a2n_ref, shift2_ref, mx_ref):
    i = pl.program_id(0)

    @pl.when(i == 0)
    def _():
        mx_ref[0] = -1e30
        mx_ref[1] = -1e30

    segs = []
    for h in range(H):
        seg = msg_ref[h // 2, :, (h % 2) * 64:(h % 2 + 1) * 64]
        dcol = den_ref[h // 4, :, (h % 4):(h % 4) + 1]
        segs.append(seg / (dcol + 1e-16))
    emb = jnp.concatenate(segs, axis=1) + b1_ref[...]
    emb_ref[...] = emb
    h2 = jnp.where(emb > 0, emb, jnp.exp(emb) - 1.0)
    h2m = lax.dot_general(h2, w2_ref[...], (((1,), (0,)), ((), ())),
                          precision=_HIGH, preferred_element_type=F32)
    tab2_ref[...] = h2m
    a_s = jnp.sum(h2m * as2_ref[...], axis=-1, keepdims=True)
    a_d = jnp.sum(h2m * ad2_ref[...], axis=-1, keepdims=True)
    col = lax.broadcasted_iota(I32, (a_s.shape[0], 16), 1)
    a2n_ref[...] = jnp.where(col == 0, a_s, jnp.where(col == 1, a_d, 0.0))
    mx_ref[0] = jnp.maximum(mx_ref[0], jnp.max(a_s))
    mx_ref[1] = jnp.maximum(mx_ref[1], jnp.max(a_d))
    shift2_ref[...] = jnp.full((8, 16), jnp.maximum(mx_ref[0] + mx_ref[1], 0.0), F32)


def _tc_mid(msg1, den1, b1_2d, W2, att_src2, att_dst2):
    BN = 256
    grid = (N1 // BN,)
    return pl.pallas_call(
        _tc_mid_body,
        grid=grid,
        in_specs=[
            pl.BlockSpec((4, BN, 128), lambda i: (0, i, 0)),
            pl.BlockSpec((2, BN, 16), lambda i: (0, i, 0)),
            pl.BlockSpec((1, H * HC), lambda i: (0, 0)),
            pl.BlockSpec((H * HC, NC), lambda i: (0, 0)),
            pl.BlockSpec((1, NC), lambda i: (0, 0)),
            pl.BlockSpec((1, NC), lambda i: (0, 0)),
        ],
        out_specs=[
            pl.BlockSpec((BN, H * HC), lambda i: (i, 0)),
            pl.BlockSpec((BN, NC), lambda i: (i, 0)),
            pl.BlockSpec((BN, 16), lambda i: (i, 0)),
            pl.BlockSpec((8, 16), lambda i: (0, 0)),
        ],
        out_shape=[
            jax.ShapeDtypeStruct((N1, H * HC), F32),
            jax.ShapeDtypeStruct((N1, NC), F32),
            jax.ShapeDtypeStruct((N1, 16), F32),
            jax.ShapeDtypeStruct((8, 16), F32),
        ],
        scratch_shapes=[pltpu.SMEM((2,), F32)],
    )(msg1, den1, b1_2d, W2, att_src2, att_dst2)


def _sc2_body(tab_hbm, an_hbm, shift_hbm, src_hbm, dst_hbm,
              msg_out, den_out,
              shift_v, src_v0, dst_v0, asg_v0, adg_v0, w16_v0, msg_v0,
              src_v1, dst_v1, asg_v1, adg_v1, w16_v1, msg_v1,
              zbuf, zbuf16,
              acc_msg, acc_den, sem, sem2, semP, semA):
    c = lax.axis_index("c")
    s = lax.axis_index("s")
    srcs = (src_v0, src_v1)
    dsts = (dst_v0, dst_v1)
    asgs = (asg_v0, asg_v1)
    adgs = (adg_v0, adg_v1)
    w16s = (w16_v0, w16_v1)
    msgs = (msg_v0, msg_v1)
    epw = ET // 32
    nchunks = epw // C2

    pltpu.sync_copy(shift_hbm, shift_v)
    shift = shift_v[...]

    @pl.loop(0, 16)
    def _(i):
        for k in range(4):
            zbuf[i, pl.ds(k * 16, 16)] = jnp.zeros((16,), F32)
        zbuf16[i, pl.ds(0, 16)] = jnp.zeros((16,), F32)

    for b in range(2):
        @pl.loop(0, C2)
        def _(i):
            w16s[b][i, pl.ds(0, 16)] = jnp.zeros((16,), F32)

    r0 = s * RPW
    for k in range(RPW // 16):
        pltpu.sync_copy(zbuf, acc_msg.at[pl.ds(r0 + k * 16, 16)])
        pltpu.sync_copy(zbuf16, acc_den.at[pl.ds(r0 + k * 16, 16)])

    plsc.subcore_barrier()

    e0 = c * (ET // 2) + s * epw
    hp1 = pltpu.async_copy(src_hbm.at[pl.ds(e0, C2)], srcs[0], semP)
    hp2 = pltpu.async_copy(dst_hbm.at[pl.ds(e0, C2)], dsts[0], semP)
    hp1.wait()
    hp2.wait()
    pltpu.async_copy(tab_hbm.at[srcs[0]], msgs[0], sem)
    hp3 = pltpu.async_copy(an_hbm.at[srcs[0]], asgs[0], sem2)
    hp4 = pltpu.async_copy(an_hbm.at[dsts[0]], adgs[0], sem2)
    hp3.wait()
    hp4.wait()

    @pl.loop(0, nchunks, step=2)
    def _(tl):
        for b in range(2):
            ob = 1 - b
            t = tl + b
            src_v, dst_v = srcs[b], dsts[b]
            asg_v, adg_v = asgs[b], adgs[b]
            w16_v, msg_v = w16s[b], msgs[b]

            nbase = c * (ET // 2) + s * epw + (t + 1) * C2
            hp1 = pltpu.async_copy(src_hbm.at[pl.ds(nbase, C2)], srcs[ob], semP)
            hp2 = pltpu.async_copy(dst_hbm.at[pl.ds(nbase, C2)], dsts[ob], semP)

            @plsc.parallel_loop(0, C2, step=16, unroll=2)
            def _(g):
                rows = lax.iota(I32, 16) + g
                a0 = (plsc.load_gather(asg_v, [rows, jnp.full((16,), 0, I32)])
                      + plsc.load_gather(adg_v, [rows, jnp.full((16,), 1, I32)]))
                a0 = jnp.maximum(a0, 0.0) + 0.2 * jnp.minimum(a0, 0.0)
                w0 = jnp.exp(a0 - shift)
                plsc.store_scatter(w16_v, [rows, jnp.full((16,), 0, I32)], w0)

            pltpu.make_async_copy(tab_hbm.at[src_v], msg_v, sem).wait()

            @plsc.parallel_loop(0, C2, unroll=4)
            def _(i):
                wrow = w16_v[i, pl.ds(0, 16)]
                v0 = jnp.full((16,), wrow[0], F32)
                for k in range(4):
                    msg_v[i, pl.ds(k * 16, 16)] = msg_v[i, pl.ds(k * 16, 16)] * v0

            h5 = pltpu.async_copy(msg_v, acc_msg.at[dst_v], semA, add=True)
            h6 = pltpu.async_copy(w16_v, acc_den.at[dst_v], semA, add=True)

            hp1.wait()
            hp2.wait()
            pltpu.async_copy(tab_hbm.at[srcs[ob]], msgs[ob], sem)
            h7 = pltpu.async_copy(an_hbm.at[srcs[ob]], asgs[ob], sem2)
            h8 = pltpu.async_copy(an_hbm.at[dsts[ob]], adgs[ob], sem2)
            h7.wait()
            h8.wait()
            h5.wait()
            h6.wait()

    pltpu.make_async_copy(tab_hbm.at[srcs[0]], msgs[0], sem).wait()

    plsc.subcore_barrier()
    pltpu.sync_copy(acc_msg.at[pl.ds(r0, RPW)],
                    msg_out.at[pl.ds(c * N1 + r0, RPW)])
    pltpu.sync_copy(acc_den.at[pl.ds(r0, RPW)],
                    den_out.at[pl.ds(c * N1 + r0, RPW)])


def _sc_layer2(tab2, a2_nodes, shift2_rep, srcE, dstE):
    mesh = plsc.VectorSubcoreMesh(core_axis_name="c", subcore_axis_name="s")
    fn = functools.partial(
        pl.kernel,
        out_type=[
            jax.ShapeDtypeStruct((2 * N1, 64), F32),
            jax.ShapeDtypeStruct((2 * N1, 16), F32),
        ],
        mesh=mesh,
        scratch_types=[
            pltpu.VMEM((16,), F32),
            pltpu.VMEM((C2,), I32),
            pltpu.VMEM((C2,), I32),
            pltpu.VMEM((C2, 16), F32),
            pltpu.VMEM((C2, 16), F32),
            pltpu.VMEM((C2, 16), F32),
            pltpu.VMEM((C2, 64), F32),
            pltpu.VMEM((C2,), I32),
            pltpu.VMEM((C2,), I32),
            pltpu.VMEM((C2, 16), F32),
            pltpu.VMEM((C2, 16), F32),
            pltpu.VMEM((C2, 16), F32),
            pltpu.VMEM((C2, 64), F32),
            pltpu.VMEM((16, 64), F32),
            pltpu.VMEM((16, 16), F32),
            pltpu.VMEM_SHARED((N1, 64), F32),
            pltpu.VMEM_SHARED((N1, 16), F32),
            pltpu.SemaphoreType.DMA,
            pltpu.SemaphoreType.DMA,
            pltpu.SemaphoreType.DMA,
            pltpu.SemaphoreType.DMA,
        ],
        compiler_params=_sc_compiler_params(),
    )(_sc2_body)
    return fn(tab2, a2_nodes, shift2_rep, srcE, dstE)


def _tc_final_body(msg_ref, den_ref, b2_ref, out_ref):
    d = den_ref[0, :, 0:1] + den_ref[1, :, 0:1] + 1e-16
    out_ref[...] = (msg_ref[0] + msg_ref[1]) / d + b2_ref[...]


def _tc_final(msg2, den2, b2_2d):
    BN = 512
    grid = (N1 // BN,)
    return pl.pallas_call(
        _tc_final_body,
        grid=grid,
        in_specs=[
            pl.BlockSpec((2, BN, NC), lambda i: (0, i, 0)),
            pl.BlockSpec((2, BN, 16), lambda i: (0, i, 0)),
            pl.BlockSpec((1, NC), lambda i: (0, 0)),
        ],
        out_specs=pl.BlockSpec((BN, NC), lambda i: (i, 0)),
        out_shape=jax.ShapeDtypeStruct((N1, NC), F32),
    )(msg2, den2, b2_2d)


def kernel(x, edge_index, W1, att_src1, att_dst1, b1, W2, att_src2, att_dst2, b2):
    src = edge_index[0]
    dst = edge_index[1]
    loop = jnp.arange(N, dtype=src.dtype)
    fill = jnp.full((ET - E0 - N + 256,), N, dtype=src.dtype)
    srcE = jnp.concatenate([src, loop, fill])
    dstE = jnp.concatenate([dst, loop, fill])

    x_p = jnp.pad(x, ((0, N1 - N), (0, 0)))

    tab1, a_nodes, shift = _tc_prep(x_p, W1, att_src1, att_dst1)
    tab1f = tab1.reshape(4 * N1, 128)
    shift_rep = shift[0]

    msg1, den1 = _sc_layer1(tab1f, a_nodes, shift_rep, srcE, dstE)
    msg1 = msg1.reshape(4, N1, 128)
    den1 = den1.reshape(2, N1, 16)

    emb_p, tab2, a2_nodes, shift2 = _tc_mid(msg1, den1, b1.reshape(1, H * HC),
                                            W2, att_src2, att_dst2)
    msg2, den2 = _sc_layer2(tab2, a2_nodes, shift2[0], srcE, dstE)
    logits_p = _tc_final(msg2.reshape(2, N1, 64), den2.reshape(2, N1, 16),
                         b2.reshape(1, NC))
    return (logits_p[:N], emb_p[:N])
